# Initial kernel scaffold; baseline (speedup 1.0000x reference)
import functools

import jax
import jax.numpy as jnp
from jax import lax
from jax.experimental import pallas as pl
from jax.experimental.pallas import tpu as pltpu
from jax.experimental.pallas import tpu_sc as plsc

N = 100000
E = 6400000
G = 128
F = 16

NC = 2
NS = 16
NW = NC * NS
CH = 128
NCHUNKS = E // CH
NFULL = NCHUNKS // NW
NREM = NCHUNKS - NFULL * NW
RPT = N // NS

_mesh = plsc.VectorSubcoreMesh(core_axis_name="c", subcore_axis_name="s")


def _sc_degree(dst, zeros, ones):
    @functools.partial(
        pl.kernel,
        out_type=jax.ShapeDtypeStruct((NC, N, F), jnp.float32),
        mesh=_mesh,
        scratch_types=[
            pltpu.VMEM((CH,), jnp.int32),
            pltpu.VMEM((CH, F), jnp.float32),
            pltpu.VMEM_SHARED((N, F), jnp.float32),
        ],
    )
    def k(dst_hbm, zeros_hbm, ones_hbm, out_hbm, idx_v, ones_v, acc):
        c = lax.axis_index("c")
        s = lax.axis_index("s")
        wid = c * NS + s
        pltpu.sync_copy(zeros_hbm, acc.at[pl.ds(s * RPT, RPT)])
        pltpu.sync_copy(ones_hbm, ones_v)
        plsc.subcore_barrier()

        def do_chunk(off):
            pltpu.sync_copy(dst_hbm.at[pl.ds(off, CH)], idx_v)
            pltpu.sync_copy(ones_v, acc.at[idx_v], add=True)

        @pl.loop(0, NFULL)
        def _(i):
            do_chunk((wid + NW * i) * CH)

        @pl.when(wid < NREM)
        def _():
            do_chunk((wid + NW * NFULL) * CH)

        plsc.subcore_barrier()
        pltpu.sync_copy(acc.at[pl.ds(s * RPT, RPT)],
                        out_hbm.at[c, pl.ds(s * RPT, RPT)])

    return k(dst, zeros, ones)


def _sc_propagate(src, dst, z, zeros):
    @functools.partial(
        pl.kernel,
        out_type=jax.ShapeDtypeStruct((NC, N, F), jnp.float32),
        mesh=_mesh,
        scratch_types=[
            pltpu.VMEM((CH,), jnp.int32),
            pltpu.VMEM((CH,), jnp.int32),
            pltpu.VMEM((CH, F), jnp.float32),
            pltpu.VMEM_SHARED((N, F), jnp.float32),
        ],
    )
    def k(src_hbm, dst_hbm, z_hbm, zeros_hbm, out_hbm, sidx, didx, rows, acc):
        c = lax.axis_index("c")
        s = lax.axis_index("s")
        wid = c * NS + s
        pltpu.sync_copy(zeros_hbm, acc.at[pl.ds(s * RPT, RPT)])
        plsc.subcore_barrier()

        def do_chunk(off):
            pltpu.sync_copy(src_hbm.at[pl.ds(off, CH)], sidx)
            pltpu.sync_copy(dst_hbm.at[pl.ds(off, CH)], didx)
            pltpu.sync_copy(z_hbm.at[sidx], rows)
            pltpu.sync_copy(rows, acc.at[didx], add=True)

        @pl.loop(0, NFULL)
        def _(i):
            do_chunk((wid + NW * i) * CH)

        @pl.when(wid < NREM)
        def _():
            do_chunk((wid + NW * NFULL) * CH)

        plsc.subcore_barrier()
        pltpu.sync_copy(acc.at[pl.ds(s * RPT, RPT)],
                        out_hbm.at[c, pl.ds(s * RPT, RPT)])

    return k(src, dst, z, zeros)


BLK = 5000
NB = N // BLK


def _tc_a_body(h0_ref, h1_ref, x_ref, w1_ref, z1_ref, dinv_ref):
    deg = h0_ref[...] + h1_ref[...] + 1.0
    dinv = lax.rsqrt(deg)
    dinv_ref[...] = dinv
    z1_ref[...] = dinv * jnp.dot(x_ref[...], w1_ref[...],
                                 preferred_element_type=jnp.float32)


def _tc_a(h0, h1, x, w1):
    return pl.pallas_call(
        _tc_a_body,
        grid=(NB,),
        in_specs=[
            pl.BlockSpec((BLK, F), lambda i: (i, 0)),
            pl.BlockSpec((BLK, F), lambda i: (i, 0)),
            pl.BlockSpec((BLK, 8), lambda i: (i, 0)),
            pl.BlockSpec((8, F), lambda i: (0, 0)),
        ],
        out_specs=[
            pl.BlockSpec((BLK, F), lambda i: (i, 0)),
            pl.BlockSpec((BLK, F), lambda i: (i, 0)),
        ],
        out_shape=[
            jax.ShapeDtypeStruct((N, F), jnp.float32),
            jax.ShapeDtypeStruct((N, F), jnp.float32),
        ],
    )(h0, h1, x, w1)


def _tc_b_body(s0_ref, s1_ref, z1_ref, dinv_ref, b1_ref, z2_ref):
    dinv = dinv_ref[...]
    h1 = dinv * (s0_ref[...] + s1_ref[...] + z1_ref[...]) + b1_ref[...]
    z2_ref[...] = dinv * jnp.maximum(h1, 0.0)


def _tc_b(s0, s1, z1, dinv, b1):
    return pl.pallas_call(
        _tc_b_body,
        grid=(NB,),
        in_specs=[
            pl.BlockSpec((BLK, F), lambda i: (i, 0)),
            pl.BlockSpec((BLK, F), lambda i: (i, 0)),
            pl.BlockSpec((BLK, F), lambda i: (i, 0)),
            pl.BlockSpec((BLK, F), lambda i: (i, 0)),
            pl.BlockSpec((1, F), lambda i: (0, 0)),
        ],
        out_specs=pl.BlockSpec((BLK, F), lambda i: (i, 0)),
        out_shape=jax.ShapeDtypeStruct((N, F), jnp.float32),
    )(s0, s1, z1, dinv, b1)


def _tc_c_body(s0_ref, s1_ref, z2_ref, dinv_ref, w2_ref, b2_ref, batch_ref,
               wfc_ref, bfc_ref, out_ref, sums_ref, cnt_ref):
    i = pl.program_id(0)

    @pl.when(i == 0)
    def _():
        sums_ref[...] = jnp.zeros_like(sums_ref)
        cnt_ref[...] = jnp.zeros_like(cnt_ref)

    t = dinv_ref[...] * (s0_ref[...] + s1_ref[...] + z2_ref[...])
    h2 = jnp.maximum(jnp.dot(t, w2_ref[...], preferred_element_type=jnp.float32)
                     + b2_ref[...], 0.0)
    b = batch_ref[0, 0, :]
    mask = (b[:, None] == lax.broadcasted_iota(jnp.int32, (BLK, G), 1)
            ).astype(jnp.float32)
    sums_ref[...] += lax.dot_general(mask, h2, (((0,), (0,)), ((), ())),
                                     preferred_element_type=jnp.float32)
    cnt_ref[...] += jnp.sum(mask, axis=0, keepdims=True)

    @pl.when(i == NB - 1)
    def _():
        pooled = sums_ref[...] / jnp.maximum(cnt_ref[0, :], 1.0)[:, None]
        logits = jnp.dot(pooled, wfc_ref[...],
                         preferred_element_type=jnp.float32) + bfc_ref[...]
        out_ref[...] = jax.nn.log_softmax(logits, axis=1)


def _tc_c(s0, s1, z2, dinv, w2, b2, batch3, wfc, bfc):
    return pl.pallas_call(
        _tc_c_body,
        grid=(NB,),
        in_specs=[
            pl.BlockSpec((BLK, F), lambda i: (i, 0)),
            pl.BlockSpec((BLK, F), lambda i: (i, 0)),
            pl.BlockSpec((BLK, F), lambda i: (i, 0)),
            pl.BlockSpec((BLK, F), lambda i: (i, 0)),
            pl.BlockSpec((F, 32), lambda i: (0, 0)),
            pl.BlockSpec((1, 32), lambda i: (0, 0)),
            pl.BlockSpec((1, 1, BLK), lambda i: (i, 0, 0)),
            pl.BlockSpec((32, 2), lambda i: (0, 0)),
            pl.BlockSpec((1, 2), lambda i: (0, 0)),
        ],
        out_specs=pl.BlockSpec((G, 2), lambda i: (0, 0)),
        out_shape=jax.ShapeDtypeStruct((G, 2), jnp.float32),
        scratch_shapes=[
            pltpu.VMEM((G, 32), jnp.float32),
            pltpu.VMEM((1, G), jnp.float32),
        ],
    )(s0, s1, z2, dinv, w2, b2, batch3, wfc, bfc)


def kernel(x, edge_index, batch, W1, b1, W2, b2, Wfc, bfc):
    src = edge_index[0]
    dst = edge_index[1]
    zeros = jnp.zeros((RPT, F), jnp.float32)
    ones = jnp.ones((CH, F), jnp.float32)

    hp = _sc_degree(dst, zeros, ones)
    z1, dinv = _tc_a(hp[0], hp[1], x, W1)
    s1 = _sc_propagate(src, dst, z1, zeros)
    z2 = _tc_b(s1[0], s1[1], z1, dinv, b1.reshape(1, F))
    s2 = _sc_propagate(src, dst, z2, zeros)
    out = _tc_c(s2[0], s2[1], z2, dinv, W2, b2.reshape(1, 32),
                batch.reshape(NB, 1, BLK), Wfc, bfc.reshape(1, 2))
    return out


# speedup vs baseline: 22.3767x; 22.3767x over previous
import functools

import jax
import jax.numpy as jnp
from jax import lax
from jax.experimental import pallas as pl
from jax.experimental.pallas import tpu as pltpu
from jax.experimental.pallas import tpu_sc as plsc

N = 100000
E = 6400000
G = 128
F = 16

NC = 2
NS = 16
NW = NC * NS
CH = 128
NCHUNKS = E // CH
NFULL = NCHUNKS // NW
NREM = NCHUNKS - NFULL * NW
RPT = 6256
RPT_LAST = N - (NS - 1) * RPT
LAST_START = (NS - 1) * RPT

_mesh = plsc.VectorSubcoreMesh(core_axis_name="c", subcore_axis_name="s")
_sc_params = pltpu.CompilerParams(use_tc_tiling_on_sc=False)


def _zero_acc(zeros_hbm, acc, s):
    @pl.when(s < NS - 1)
    def _():
        pltpu.sync_copy(zeros_hbm, acc.at[pl.ds(s * RPT, RPT)])

    @pl.when(s == NS - 1)
    def _():
        pltpu.sync_copy(zeros_hbm.at[pl.ds(0, RPT_LAST)],
                        acc.at[pl.ds(LAST_START, RPT_LAST)])


def _flush_acc(acc, out_hbm, c, s):
    @pl.when(s < NS - 1)
    def _():
        pltpu.sync_copy(acc.at[pl.ds(s * RPT, RPT)],
                        out_hbm.at[c, pl.ds(s * RPT, RPT)])

    @pl.when(s == NS - 1)
    def _():
        pltpu.sync_copy(acc.at[pl.ds(LAST_START, RPT_LAST)],
                        out_hbm.at[c, pl.ds(LAST_START, RPT_LAST)])


def _sc_degree(dst, zeros, ones):
    @functools.partial(
        pl.kernel,
        out_type=jax.ShapeDtypeStruct((NC, N, F), jnp.float32),
        mesh=_mesh,
        compiler_params=_sc_params,
        scratch_types=[
            pltpu.VMEM((CH,), jnp.int32),
            pltpu.VMEM((CH, F), jnp.float32),
            pltpu.VMEM_SHARED((N, F), jnp.float32),
        ],
    )
    def k(dst_hbm, zeros_hbm, ones_hbm, out_hbm, idx_v, ones_v, acc):
        c = lax.axis_index("c")
        s = lax.axis_index("s")
        wid = c * NS + s
        _zero_acc(zeros_hbm, acc, s)
        pltpu.sync_copy(ones_hbm, ones_v)
        plsc.subcore_barrier()

        def do_chunk(off):
            pltpu.sync_copy(dst_hbm.at[pl.ds(off, CH)], idx_v)
            pltpu.sync_copy(ones_v, acc.at[idx_v], add=True)

        @pl.loop(0, NFULL)
        def _(i):
            do_chunk((wid + NW * i) * CH)

        @pl.when(wid < NREM)
        def _():
            do_chunk((wid + NW * NFULL) * CH)

        plsc.subcore_barrier()
        _flush_acc(acc, out_hbm, c, s)

    return k(dst, zeros, ones)


def _sc_propagate(src, dst, z, zeros):
    @functools.partial(
        pl.kernel,
        out_type=jax.ShapeDtypeStruct((NC, N, F), jnp.float32),
        mesh=_mesh,
        compiler_params=_sc_params,
        scratch_types=[
            pltpu.VMEM((CH,), jnp.int32),
            pltpu.VMEM((CH,), jnp.int32),
            pltpu.VMEM((CH, F), jnp.float32),
            pltpu.VMEM_SHARED((N, F), jnp.float32),
        ],
    )
    def k(src_hbm, dst_hbm, z_hbm, zeros_hbm, out_hbm, sidx, didx, rows, acc):
        c = lax.axis_index("c")
        s = lax.axis_index("s")
        wid = c * NS + s
        _zero_acc(zeros_hbm, acc, s)
        plsc.subcore_barrier()

        def do_chunk(off):
            pltpu.sync_copy(src_hbm.at[pl.ds(off, CH)], sidx)
            pltpu.sync_copy(dst_hbm.at[pl.ds(off, CH)], didx)
            pltpu.sync_copy(z_hbm.at[sidx], rows)
            pltpu.sync_copy(rows, acc.at[didx], add=True)

        @pl.loop(0, NFULL)
        def _(i):
            do_chunk((wid + NW * i) * CH)

        @pl.when(wid < NREM)
        def _():
            do_chunk((wid + NW * NFULL) * CH)

        plsc.subcore_barrier()
        _flush_acc(acc, out_hbm, c, s)

    return k(src, dst, z, zeros)


BLK = 5000
NB = N // BLK


def _tc_a_body(h0_ref, h1_ref, x_ref, w1_ref, z1_ref, dinv_ref):
    deg = h0_ref[...] + h1_ref[...] + 1.0
    dinv = lax.rsqrt(deg)
    dinv_ref[...] = dinv
    z1_ref[...] = dinv * jnp.dot(x_ref[...], w1_ref[...],
                                 preferred_element_type=jnp.float32)


def _tc_a(h0, h1, x, w1):
    return pl.pallas_call(
        _tc_a_body,
        grid=(NB,),
        in_specs=[
            pl.BlockSpec((BLK, F), lambda i: (i, 0)),
            pl.BlockSpec((BLK, F), lambda i: (i, 0)),
            pl.BlockSpec((BLK, 8), lambda i: (i, 0)),
            pl.BlockSpec((8, F), lambda i: (0, 0)),
        ],
        out_specs=[
            pl.BlockSpec((BLK, F), lambda i: (i, 0)),
            pl.BlockSpec((BLK, F), lambda i: (i, 0)),
        ],
        out_shape=[
            jax.ShapeDtypeStruct((N, F), jnp.float32),
            jax.ShapeDtypeStruct((N, F), jnp.float32),
        ],
    )(h0, h1, x, w1)


def _tc_b_body(s0_ref, s1_ref, z1_ref, dinv_ref, b1_ref, z2_ref):
    dinv = dinv_ref[...]
    h1 = dinv * (s0_ref[...] + s1_ref[...] + z1_ref[...]) + b1_ref[...]
    z2_ref[...] = dinv * jnp.maximum(h1, 0.0)


def _tc_b(s0, s1, z1, dinv, b1):
    return pl.pallas_call(
        _tc_b_body,
        grid=(NB,),
        in_specs=[
            pl.BlockSpec((BLK, F), lambda i: (i, 0)),
            pl.BlockSpec((BLK, F), lambda i: (i, 0)),
            pl.BlockSpec((BLK, F), lambda i: (i, 0)),
            pl.BlockSpec((BLK, F), lambda i: (i, 0)),
            pl.BlockSpec((1, F), lambda i: (0, 0)),
        ],
        out_specs=pl.BlockSpec((BLK, F), lambda i: (i, 0)),
        out_shape=jax.ShapeDtypeStruct((N, F), jnp.float32),
    )(s0, s1, z1, dinv, b1)


def _tc_c_body(s0_ref, s1_ref, z2_ref, dinv_ref, w2_ref, b2_ref, batch_ref,
               wfc_ref, bfc_ref, out_ref, sums_ref, cnt_ref):
    i = pl.program_id(0)

    @pl.when(i == 0)
    def _():
        sums_ref[...] = jnp.zeros_like(sums_ref)
        cnt_ref[...] = jnp.zeros_like(cnt_ref)

    t = dinv_ref[...] * (s0_ref[...] + s1_ref[...] + z2_ref[...])
    h2 = jnp.maximum(jnp.dot(t, w2_ref[...], preferred_element_type=jnp.float32)
                     + b2_ref[...], 0.0)
    b = batch_ref[0, 0, :]
    mask = (b[:, None] == lax.broadcasted_iota(jnp.int32, (BLK, G), 1)
            ).astype(jnp.float32)
    sums_ref[...] += lax.dot_general(mask, h2, (((0,), (0,)), ((), ())),
                                     preferred_element_type=jnp.float32)
    cnt_ref[...] += jnp.sum(mask, axis=0, keepdims=True)

    @pl.when(i == NB - 1)
    def _():
        pooled = sums_ref[...] / jnp.maximum(cnt_ref[0, :], 1.0)[:, None]
        logits = jnp.dot(pooled, wfc_ref[...],
                         preferred_element_type=jnp.float32) + bfc_ref[...]
        out_ref[...] = jax.nn.log_softmax(logits, axis=1)


def _tc_c(s0, s1, z2, dinv, w2, b2, batch3, wfc, bfc):
    return pl.pallas_call(
        _tc_c_body,
        grid=(NB,),
        in_specs=[
            pl.BlockSpec((BLK, F), lambda i: (i, 0)),
            pl.BlockSpec((BLK, F), lambda i: (i, 0)),
            pl.BlockSpec((BLK, F), lambda i: (i, 0)),
            pl.BlockSpec((BLK, F), lambda i: (i, 0)),
            pl.BlockSpec((F, 32), lambda i: (0, 0)),
            pl.BlockSpec((1, 32), lambda i: (0, 0)),
            pl.BlockSpec((1, 1, BLK), lambda i: (i, 0, 0)),
            pl.BlockSpec((32, 2), lambda i: (0, 0)),
            pl.BlockSpec((1, 2), lambda i: (0, 0)),
        ],
        out_specs=pl.BlockSpec((G, 2), lambda i: (0, 0)),
        out_shape=jax.ShapeDtypeStruct((G, 2), jnp.float32),
        scratch_shapes=[
            pltpu.VMEM((G, 32), jnp.float32),
            pltpu.VMEM((1, G), jnp.float32),
        ],
    )(s0, s1, z2, dinv, w2, b2, batch3, wfc, bfc)


def kernel(x, edge_index, batch, W1, b1, W2, b2, Wfc, bfc):
    src = edge_index[0]
    dst = edge_index[1]
    zeros = jnp.zeros((RPT, F), jnp.float32)
    ones = jnp.ones((CH, F), jnp.float32)

    hp = _sc_degree(dst, zeros, ones)
    z1, dinv = _tc_a(hp[0], hp[1], x, W1)
    s1 = _sc_propagate(src, dst, z1, zeros)
    z2 = _tc_b(s1[0], s1[1], z1, dinv, b1.reshape(1, F))
    s2 = _sc_propagate(src, dst, z2, zeros)
    out = _tc_c(s2[0], s2[1], z2, dinv, W2, b2.reshape(1, 32),
                batch.reshape(NB, 1, BLK), Wfc, bfc.reshape(1, 2))
    return out


# speedup vs baseline: 88.1863x; 3.9410x over previous
import functools

import jax
import jax.numpy as jnp
from jax import lax
from jax.experimental import pallas as pl
from jax.experimental.pallas import tpu as pltpu
from jax.experimental.pallas import tpu_sc as plsc

N = 100000
E = 6400000
G = 128
F = 16

NC = 2
NS = 16
NW = NC * NS
CH = 128
NCHUNKS = E // CH
NFULL = NCHUNKS // NW
NREM = NCHUNKS - NFULL * NW
S = 5
NSUPER = 312
NPIPE = S * NSUPER
NTAIL = NFULL - NPIPE
RPT = 6256
RPT_LAST = N - (NS - 1) * RPT
LAST_START = (NS - 1) * RPT

_mesh = plsc.VectorSubcoreMesh(core_axis_name="c", subcore_axis_name="s")
_sc_params = pltpu.CompilerParams(use_tc_tiling_on_sc=False)


def _zero_acc(zeros_hbm, acc, s):
    @pl.when(s < NS - 1)
    def _():
        pltpu.sync_copy(zeros_hbm, acc.at[pl.ds(s * RPT, RPT)])

    @pl.when(s == NS - 1)
    def _():
        pltpu.sync_copy(zeros_hbm.at[pl.ds(0, RPT_LAST)],
                        acc.at[pl.ds(LAST_START, RPT_LAST)])


def _flush_acc(acc, out_hbm, c, s):
    @pl.when(s < NS - 1)
    def _():
        pltpu.sync_copy(acc.at[pl.ds(s * RPT, RPT)],
                        out_hbm.at[c, pl.ds(s * RPT, RPT)])

    @pl.when(s == NS - 1)
    def _():
        pltpu.sync_copy(acc.at[pl.ds(LAST_START, RPT_LAST)],
                        out_hbm.at[c, pl.ds(LAST_START, RPT_LAST)])


def _sc_degree(dst2, zeros, ones):
    @functools.partial(
        pl.kernel,
        out_type=jax.ShapeDtypeStruct((NC, N, F), jnp.float32),
        mesh=_mesh,
        compiler_params=_sc_params,
        scratch_types=[
            pltpu.VMEM((S, CH), jnp.int32),
            pltpu.VMEM((S, CH), jnp.int32),
            pltpu.VMEM((CH, F), jnp.float32),
            pltpu.VMEM_SHARED((N, F), jnp.float32),
            pltpu.SemaphoreType.DMA,
            pltpu.SemaphoreType.DMA,
            pltpu.SemaphoreType.DMA,
            pltpu.SemaphoreType.DMA,
        ],
    )
    def k(dst_hbm, zeros_hbm, ones_hbm, out_hbm,
          didx0, didx1, ones_v, acc, dsem0, dsem1, ssem0, ssem1):
        didx = (didx0, didx1)
        dsem = (dsem0, dsem1)
        ssem = (ssem0, ssem1)
        c = lax.axis_index("c")
        s = lax.axis_index("s")
        wid = c * NS + s
        cw = wid * NFULL + jnp.minimum(wid, NREM)
        _zero_acc(zeros_hbm, acc, s)
        pltpu.sync_copy(ones_hbm, ones_v)
        plsc.subcore_barrier()

        pltpu.async_copy(dst_hbm.at[pl.ds(cw, S)], didx0, dsem0)

        def body(i, p, np):
            pltpu.make_async_copy(dst_hbm.at[pl.ds(cw + i * S, S)],
                                  didx[p], dsem[p]).wait()

            @pl.when(i > 0)
            def _():
                for j in range(S):
                    pltpu.make_async_copy(ones_v, acc.at[didx[np].at[j]],
                                          ssem[np]).wait()

            @pl.when(i + 1 < NSUPER)
            def _():
                pltpu.async_copy(dst_hbm.at[pl.ds(cw + (i + 1) * S, S)],
                                 didx[np], dsem[np])

            for j in range(S):
                pltpu.async_copy(ones_v, acc.at[didx[p].at[j]], ssem[p],
                                 add=True)

        @pl.loop(0, NSUPER, step=2)
        def _(i):
            body(i, 0, 1)
            body(i + 1, 1, 0)

        for j in range(S):
            pltpu.make_async_copy(ones_v, acc.at[didx1.at[j]], ssem1).wait()

        def tail_chunk(t):
            pltpu.sync_copy(dst_hbm.at[pl.ds(cw + NPIPE + t, 1)],
                            didx0.at[pl.ds(0, 1)])
            pltpu.sync_copy(ones_v, acc.at[didx0.at[0]], add=True)

        for t in range(NTAIL):
            tail_chunk(t)

        @pl.when(wid < NREM)
        def _():
            tail_chunk(NTAIL)

        plsc.subcore_barrier()
        _flush_acc(acc, out_hbm, c, s)

    return k(dst2, zeros, ones)


def _sc_propagate(src2, dst2, z, zeros):
    @functools.partial(
        pl.kernel,
        out_type=jax.ShapeDtypeStruct((NC, N, F), jnp.float32),
        mesh=_mesh,
        compiler_params=_sc_params,
        scratch_types=[
            pltpu.VMEM((S, CH), jnp.int32),
            pltpu.VMEM((S, CH), jnp.int32),
            pltpu.VMEM((S, CH), jnp.int32),
            pltpu.VMEM((S, CH), jnp.int32),
            pltpu.VMEM((S, CH, F), jnp.float32),
            pltpu.VMEM((S, CH, F), jnp.float32),
            pltpu.VMEM_SHARED((N, F), jnp.float32),
            pltpu.SemaphoreType.DMA,
            pltpu.SemaphoreType.DMA,
            pltpu.SemaphoreType.DMA,
            pltpu.SemaphoreType.DMA,
            pltpu.SemaphoreType.DMA,
            pltpu.SemaphoreType.DMA,
        ],
    )
    def k(src_hbm, dst_hbm, z_hbm, zeros_hbm, out_hbm,
          sidx0, sidx1, didx0, didx1, rows0, rows1, acc,
          dsem0, dsem1, gsem0, gsem1, ssem0, ssem1):
        sidx = (sidx0, sidx1)
        didx = (didx0, didx1)
        rows = (rows0, rows1)
        dsem = (dsem0, dsem1)
        gsem = (gsem0, gsem1)
        ssem = (ssem0, ssem1)
        c = lax.axis_index("c")
        s = lax.axis_index("s")
        wid = c * NS + s
        cw = wid * NFULL + jnp.minimum(wid, NREM)
        _zero_acc(zeros_hbm, acc, s)
        plsc.subcore_barrier()

        pltpu.async_copy(src_hbm.at[pl.ds(cw, S)], sidx0, dsem0)
        pltpu.async_copy(dst_hbm.at[pl.ds(cw, S)], didx0, dsem0)

        def body(i, p, np):
            pltpu.make_async_copy(src_hbm.at[pl.ds(cw + i * S, S)],
                                  sidx[p], dsem[p]).wait()
            pltpu.make_async_copy(dst_hbm.at[pl.ds(cw + i * S, S)],
                                  didx[p], dsem[p]).wait()
            gh = [pltpu.async_copy(z_hbm.at[sidx[p].at[j]], rows[p].at[j],
                                   gsem[p]) for j in range(S)]

            @pl.when(i > 0)
            def _():
                for j in range(S):
                    pltpu.make_async_copy(rows[np].at[j],
                                          acc.at[didx[np].at[j]],
                                          ssem[np]).wait()

            @pl.when(i + 1 < NSUPER)
            def _():
                pltpu.async_copy(src_hbm.at[pl.ds(cw + (i + 1) * S, S)],
                                 sidx[np], dsem[np])
                pltpu.async_copy(dst_hbm.at[pl.ds(cw + (i + 1) * S, S)],
                                 didx[np], dsem[np])

            for h in gh:
                h.wait()
            for j in range(S):
                pltpu.async_copy(rows[p].at[j], acc.at[didx[p].at[j]],
                                 ssem[p], add=True)

        @pl.loop(0, NSUPER, step=2)
        def _(i):
            body(i, 0, 1)
            body(i + 1, 1, 0)

        for j in range(S):
            pltpu.make_async_copy(rows1.at[j], acc.at[didx1.at[j]],
                                  ssem1).wait()

        def tail_chunk(t):
            pltpu.sync_copy(src_hbm.at[pl.ds(cw + NPIPE + t, 1)],
                            sidx0.at[pl.ds(0, 1)])
            pltpu.sync_copy(dst_hbm.at[pl.ds(cw + NPIPE + t, 1)],
                            didx0.at[pl.ds(0, 1)])
            pltpu.sync_copy(z_hbm.at[sidx0.at[0]], rows0.at[0])
            pltpu.sync_copy(rows0.at[0], acc.at[didx0.at[0]], add=True)

        for t in range(NTAIL):
            tail_chunk(t)

        @pl.when(wid < NREM)
        def _():
            tail_chunk(NTAIL)

        plsc.subcore_barrier()
        _flush_acc(acc, out_hbm, c, s)

    return k(src2, dst2, z, zeros)


BLK = 5000
NB = N // BLK


def _tc_a_body(h0_ref, h1_ref, x_ref, w1_ref, z1_ref, dinv_ref):
    deg = h0_ref[...] + h1_ref[...] + 1.0
    dinv = lax.rsqrt(deg)
    dinv_ref[...] = dinv
    z1_ref[...] = dinv * jnp.dot(x_ref[...], w1_ref[...],
                                 preferred_element_type=jnp.float32)


def _tc_a(h0, h1, x, w1):
    return pl.pallas_call(
        _tc_a_body,
        grid=(NB,),
        in_specs=[
            pl.BlockSpec((BLK, F), lambda i: (i, 0)),
            pl.BlockSpec((BLK, F), lambda i: (i, 0)),
            pl.BlockSpec((BLK, 8), lambda i: (i, 0)),
            pl.BlockSpec((8, F), lambda i: (0, 0)),
        ],
        out_specs=[
            pl.BlockSpec((BLK, F), lambda i: (i, 0)),
            pl.BlockSpec((BLK, F), lambda i: (i, 0)),
        ],
        out_shape=[
            jax.ShapeDtypeStruct((N, F), jnp.float32),
            jax.ShapeDtypeStruct((N, F), jnp.float32),
        ],
    )(h0, h1, x, w1)


def _tc_b_body(s0_ref, s1_ref, z1_ref, dinv_ref, b1_ref, z2_ref):
    dinv = dinv_ref[...]
    h1 = dinv * (s0_ref[...] + s1_ref[...] + z1_ref[...]) + b1_ref[...]
    z2_ref[...] = dinv * jnp.maximum(h1, 0.0)


def _tc_b(s0, s1, z1, dinv, b1):
    return pl.pallas_call(
        _tc_b_body,
        grid=(NB,),
        in_specs=[
            pl.BlockSpec((BLK, F), lambda i: (i, 0)),
            pl.BlockSpec((BLK, F), lambda i: (i, 0)),
            pl.BlockSpec((BLK, F), lambda i: (i, 0)),
            pl.BlockSpec((BLK, F), lambda i: (i, 0)),
            pl.BlockSpec((1, F), lambda i: (0, 0)),
        ],
        out_specs=pl.BlockSpec((BLK, F), lambda i: (i, 0)),
        out_shape=jax.ShapeDtypeStruct((N, F), jnp.float32),
    )(s0, s1, z1, dinv, b1)


def _tc_c_body(s0_ref, s1_ref, z2_ref, dinv_ref, w2_ref, b2_ref, batch_ref,
               wfc_ref, bfc_ref, out_ref, sums_ref, cnt_ref):
    i = pl.program_id(0)

    @pl.when(i == 0)
    def _():
        sums_ref[...] = jnp.zeros_like(sums_ref)
        cnt_ref[...] = jnp.zeros_like(cnt_ref)

    t = dinv_ref[...] * (s0_ref[...] + s1_ref[...] + z2_ref[...])
    h2 = jnp.maximum(jnp.dot(t, w2_ref[...], preferred_element_type=jnp.float32)
                     + b2_ref[...], 0.0)
    b = batch_ref[0, 0, :]
    mask = (b[:, None] == lax.broadcasted_iota(jnp.int32, (BLK, G), 1)
            ).astype(jnp.float32)
    sums_ref[...] += lax.dot_general(mask, h2, (((0,), (0,)), ((), ())),
                                     preferred_element_type=jnp.float32)
    cnt_ref[...] += jnp.sum(mask, axis=0, keepdims=True)

    @pl.when(i == NB - 1)
    def _():
        pooled = sums_ref[...] / jnp.maximum(cnt_ref[0, :], 1.0)[:, None]
        logits = jnp.dot(pooled, wfc_ref[...],
                         preferred_element_type=jnp.float32) + bfc_ref[...]
        out_ref[...] = jax.nn.log_softmax(logits, axis=1)


def _tc_c(s0, s1, z2, dinv, w2, b2, batch3, wfc, bfc):
    return pl.pallas_call(
        _tc_c_body,
        grid=(NB,),
        in_specs=[
            pl.BlockSpec((BLK, F), lambda i: (i, 0)),
            pl.BlockSpec((BLK, F), lambda i: (i, 0)),
            pl.BlockSpec((BLK, F), lambda i: (i, 0)),
            pl.BlockSpec((BLK, F), lambda i: (i, 0)),
            pl.BlockSpec((F, 32), lambda i: (0, 0)),
            pl.BlockSpec((1, 32), lambda i: (0, 0)),
            pl.BlockSpec((1, 1, BLK), lambda i: (i, 0, 0)),
            pl.BlockSpec((32, 2), lambda i: (0, 0)),
            pl.BlockSpec((1, 2), lambda i: (0, 0)),
        ],
        out_specs=pl.BlockSpec((G, 2), lambda i: (0, 0)),
        out_shape=jax.ShapeDtypeStruct((G, 2), jnp.float32),
        scratch_shapes=[
            pltpu.VMEM((G, 32), jnp.float32),
            pltpu.VMEM((1, G), jnp.float32),
        ],
    )(s0, s1, z2, dinv, w2, b2, batch3, wfc, bfc)


def kernel(x, edge_index, batch, W1, b1, W2, b2, Wfc, bfc):
    src2 = edge_index[0].reshape(NCHUNKS, CH)
    dst2 = edge_index[1].reshape(NCHUNKS, CH)
    zeros = jnp.zeros((RPT, F), jnp.float32)
    ones = jnp.ones((CH, F), jnp.float32)

    hp = _sc_degree(dst2, zeros, ones)
    z1, dinv = _tc_a(hp[0], hp[1], x, W1)
    s1 = _sc_propagate(src2, dst2, z1, zeros)
    z2 = _tc_b(s1[0], s1[1], z1, dinv, b1.reshape(1, F))
    s2 = _sc_propagate(src2, dst2, z2, zeros)
    out = _tc_c(s2[0], s2[1], z2, dinv, W2, b2.reshape(1, 32),
                batch.reshape(NB, 1, BLK), Wfc, bfc.reshape(1, 2))
    return out


# speedup vs baseline: 96.2584x; 1.0915x over previous
import functools

import jax
import jax.numpy as jnp
from jax import lax
from jax.experimental import pallas as pl
from jax.experimental.pallas import tpu as pltpu
from jax.experimental.pallas import tpu_sc as plsc

N = 100000
E = 6400000
G = 128
F = 16

NC = 2
NS = 16
NW = NC * NS
CH = 128
NCHUNKS = E // CH
NFULL = NCHUNKS // NW
NREM = NCHUNKS - NFULL * NW
S = 5
NSUPER = 312
NPIPE = S * NSUPER
NTAIL = NFULL - NPIPE
RPT = 6256
RPT_LAST = N - (NS - 1) * RPT
LAST_START = (NS - 1) * RPT

_mesh = plsc.VectorSubcoreMesh(core_axis_name="c", subcore_axis_name="s")
_sc_params = pltpu.CompilerParams(use_tc_tiling_on_sc=False)


def _zero_acc(zeros_hbm, acc, s):
    @pl.when(s < NS - 1)
    def _():
        pltpu.sync_copy(zeros_hbm, acc.at[pl.ds(s * RPT, RPT)])

    @pl.when(s == NS - 1)
    def _():
        pltpu.sync_copy(zeros_hbm.at[pl.ds(0, RPT_LAST)],
                        acc.at[pl.ds(LAST_START, RPT_LAST)])


def _flush_acc(acc, out_hbm, c, s):
    @pl.when(s < NS - 1)
    def _():
        pltpu.sync_copy(acc.at[pl.ds(s * RPT, RPT)],
                        out_hbm.at[c, pl.ds(s * RPT, RPT)])

    @pl.when(s == NS - 1)
    def _():
        pltpu.sync_copy(acc.at[pl.ds(LAST_START, RPT_LAST)],
                        out_hbm.at[c, pl.ds(LAST_START, RPT_LAST)])


def _sc_degree(dst2, zeros, ones):
    @functools.partial(
        pl.kernel,
        out_type=jax.ShapeDtypeStruct((NC, N, F), jnp.float32),
        mesh=_mesh,
        compiler_params=_sc_params,
        scratch_types=[
            pltpu.VMEM((S, CH), jnp.int32),
            pltpu.VMEM((S, CH), jnp.int32),
            pltpu.VMEM((CH, F), jnp.float32),
            pltpu.VMEM_SHARED((N, F), jnp.float32),
            pltpu.SemaphoreType.DMA,
            pltpu.SemaphoreType.DMA,
            pltpu.SemaphoreType.DMA,
            pltpu.SemaphoreType.DMA,
        ],
    )
    def k(dst_hbm, zeros_hbm, ones_hbm, out_hbm,
          didx0, didx1, ones_v, acc, dsem0, dsem1, ssem0, ssem1):
        didx = (didx0, didx1)
        dsem = (dsem0, dsem1)
        ssem = (ssem0, ssem1)
        c = lax.axis_index("c")
        s = lax.axis_index("s")
        wid = c * NS + s
        cw = wid * NFULL + jnp.minimum(wid, NREM)
        _zero_acc(zeros_hbm, acc, s)
        pltpu.sync_copy(ones_hbm, ones_v)
        plsc.subcore_barrier()

        pltpu.async_copy(dst_hbm.at[pl.ds(cw, S)], didx0, dsem0)

        def body(i, p, np):
            pltpu.make_async_copy(dst_hbm.at[pl.ds(cw + i * S, S)],
                                  didx[p], dsem[p]).wait()

            @pl.when(i > 0)
            def _():
                for j in range(S):
                    pltpu.make_async_copy(ones_v, acc.at[didx[np].at[j]],
                                          ssem[np]).wait()

            @pl.when(i + 1 < NSUPER)
            def _():
                pltpu.async_copy(dst_hbm.at[pl.ds(cw + (i + 1) * S, S)],
                                 didx[np], dsem[np])

            for j in range(S):
                pltpu.async_copy(ones_v, acc.at[didx[p].at[j]], ssem[p],
                                 add=True)

        @pl.loop(0, NSUPER, step=2)
        def _(i):
            body(i, 0, 1)
            body(i + 1, 1, 0)

        for j in range(S):
            pltpu.make_async_copy(ones_v, acc.at[didx1.at[j]], ssem1).wait()

        def tail_chunk(t):
            pltpu.sync_copy(dst_hbm.at[pl.ds(cw + NPIPE + t, 1)],
                            didx0.at[pl.ds(0, 1)])
            pltpu.sync_copy(ones_v, acc.at[didx0.at[0]], add=True)

        for t in range(NTAIL):
            tail_chunk(t)

        @pl.when(wid < NREM)
        def _():
            tail_chunk(NTAIL)

        plsc.subcore_barrier()
        _flush_acc(acc, out_hbm, c, s)

    return k(dst2, zeros, ones)


def _sc_propagate(src2, dst2, z, zeros):
    @functools.partial(
        pl.kernel,
        out_type=jax.ShapeDtypeStruct((NC, N, F), jnp.float32),
        mesh=_mesh,
        compiler_params=_sc_params,
        scratch_types=[
            pltpu.VMEM((S, CH), jnp.int32),
            pltpu.VMEM((S, CH), jnp.int32),
            pltpu.VMEM((S, CH), jnp.int32),
            pltpu.VMEM((S, CH), jnp.int32),
            pltpu.VMEM((S, CH, F), jnp.float32),
            pltpu.VMEM((S, CH, F), jnp.float32),
            pltpu.VMEM_SHARED((N, F), jnp.float32),
            pltpu.SemaphoreType.DMA,
            pltpu.SemaphoreType.DMA,
            pltpu.SemaphoreType.DMA,
            pltpu.SemaphoreType.DMA,
            pltpu.SemaphoreType.DMA,
            pltpu.SemaphoreType.DMA,
        ],
    )
    def k(src_hbm, dst_hbm, z_hbm, zeros_hbm, out_hbm,
          sidx0, sidx1, didx0, didx1, rows0, rows1, acc,
          dsem0, dsem1, gsem0, gsem1, ssem0, ssem1):
        sidx = (sidx0, sidx1)
        didx = (didx0, didx1)
        rows = (rows0, rows1)
        dsem = (dsem0, dsem1)
        gsem = (gsem0, gsem1)
        ssem = (ssem0, ssem1)
        c = lax.axis_index("c")
        s = lax.axis_index("s")
        wid = c * NS + s
        cw = wid * NFULL + jnp.minimum(wid, NREM)
        _zero_acc(zeros_hbm, acc, s)
        plsc.subcore_barrier()

        pltpu.async_copy(src_hbm.at[pl.ds(cw, S)], sidx0, dsem0)
        pltpu.async_copy(dst_hbm.at[pl.ds(cw, S)], didx0, dsem0)

        def body(i, p, np):
            pltpu.make_async_copy(src_hbm.at[pl.ds(cw + i * S, S)],
                                  sidx[p], dsem[p]).wait()
            pltpu.make_async_copy(dst_hbm.at[pl.ds(cw + i * S, S)],
                                  didx[p], dsem[p]).wait()
            gh = [pltpu.async_copy(z_hbm.at[sidx[p].at[j]], rows[p].at[j],
                                   gsem[p]) for j in range(S)]

            @pl.when(i > 0)
            def _():
                for j in range(S):
                    pltpu.make_async_copy(rows[np].at[j],
                                          acc.at[didx[np].at[j]],
                                          ssem[np]).wait()

            @pl.when(i + 1 < NSUPER)
            def _():
                pltpu.async_copy(src_hbm.at[pl.ds(cw + (i + 1) * S, S)],
                                 sidx[np], dsem[np])
                pltpu.async_copy(dst_hbm.at[pl.ds(cw + (i + 1) * S, S)],
                                 didx[np], dsem[np])

            for h in gh:
                h.wait()
            for j in range(S):
                pltpu.async_copy(rows[p].at[j], acc.at[didx[p].at[j]],
                                 ssem[p], add=True)

        @pl.loop(0, NSUPER, step=2)
        def _(i):
            body(i, 0, 1)
            body(i + 1, 1, 0)

        for j in range(S):
            pltpu.make_async_copy(rows1.at[j], acc.at[didx1.at[j]],
                                  ssem1).wait()

        def tail_chunk(t):
            pltpu.sync_copy(src_hbm.at[pl.ds(cw + NPIPE + t, 1)],
                            sidx0.at[pl.ds(0, 1)])
            pltpu.sync_copy(dst_hbm.at[pl.ds(cw + NPIPE + t, 1)],
                            didx0.at[pl.ds(0, 1)])
            pltpu.sync_copy(z_hbm.at[sidx0.at[0]], rows0.at[0])
            pltpu.sync_copy(rows0.at[0], acc.at[didx0.at[0]], add=True)

        for t in range(NTAIL):
            tail_chunk(t)

        @pl.when(wid < NREM)
        def _():
            tail_chunk(NTAIL)

        plsc.subcore_barrier()
        _flush_acc(acc, out_hbm, c, s)

    return k(src2, dst2, z, zeros)


NBW = 20
RW = 625


def _tc_a_body(h0_ref, h1_ref, x_ref, w1_ref, z1_ref, dinv_ref):
    deg = h0_ref[0] + h1_ref[0] + 1.0
    dinv = lax.rsqrt(deg)
    dinv_ref[0] = dinv
    z1_ref[0] = dinv * jnp.dot(x_ref[0], w1_ref[...],
                               preferred_element_type=jnp.float32)


def _tc_a(h0, h1, xw, w1big):
    return pl.pallas_call(
        _tc_a_body,
        grid=(NBW,),
        in_specs=[
            pl.BlockSpec((1, RW, 128), lambda i: (i, 0, 0)),
            pl.BlockSpec((1, RW, 128), lambda i: (i, 0, 0)),
            pl.BlockSpec((1, RW, 64), lambda i: (i, 0, 0)),
            pl.BlockSpec((64, 128), lambda i: (0, 0)),
        ],
        out_specs=[
            pl.BlockSpec((1, RW, 128), lambda i: (i, 0, 0)),
            pl.BlockSpec((1, RW, 128), lambda i: (i, 0, 0)),
        ],
        out_shape=[
            jax.ShapeDtypeStruct((NBW, RW, 128), jnp.float32),
            jax.ShapeDtypeStruct((NBW, RW, 128), jnp.float32),
        ],
    )(h0, h1, xw, w1big)


def _tc_b_body(s0_ref, s1_ref, z1_ref, dinv_ref, b1_ref, z2_ref):
    dinv = dinv_ref[0]
    h1 = dinv * (s0_ref[0] + s1_ref[0] + z1_ref[0]) + b1_ref[...]
    z2_ref[0] = dinv * jnp.maximum(h1, 0.0)


def _tc_b(s0, s1, z1, dinv, b1big):
    return pl.pallas_call(
        _tc_b_body,
        grid=(NBW,),
        in_specs=[
            pl.BlockSpec((1, RW, 128), lambda i: (i, 0, 0)),
            pl.BlockSpec((1, RW, 128), lambda i: (i, 0, 0)),
            pl.BlockSpec((1, RW, 128), lambda i: (i, 0, 0)),
            pl.BlockSpec((1, RW, 128), lambda i: (i, 0, 0)),
            pl.BlockSpec((1, 128), lambda i: (0, 0)),
        ],
        out_specs=pl.BlockSpec((1, RW, 128), lambda i: (i, 0, 0)),
        out_shape=jax.ShapeDtypeStruct((NBW, RW, 128), jnp.float32),
    )(s0, s1, z1, dinv, b1big)


def _tc_c_body(s0_ref, s1_ref, z2_ref, dinv_ref, w2_ref, b2_ref, batch_ref,
               wfc_ref, bfc_ref, out_ref, sums_ref, cnt_ref):
    i = pl.program_id(0)

    @pl.when(i == 0)
    def _():
        sums_ref[...] = jnp.zeros_like(sums_ref)
        cnt_ref[...] = jnp.zeros_like(cnt_ref)

    tw = dinv_ref[0] * (s0_ref[0] + s1_ref[0] + z2_ref[0])
    h2w = jnp.maximum(jnp.dot(tw, w2_ref[...],
                              preferred_element_type=jnp.float32)
                      + b2_ref[...], 0.0)
    bt = batch_ref[0]
    for a in range(8):
        mask = (bt[:, a][:, None] == lax.broadcasted_iota(jnp.int32, (RW, G), 1)
                ).astype(jnp.float32)
        sums_ref[...] += lax.dot_general(
            mask, h2w[:, 32 * a:32 * a + 32], (((0,), (0,)), ((), ())),
            preferred_element_type=jnp.float32)
        cnt_ref[...] += jnp.sum(mask, axis=0, keepdims=True)

    @pl.when(i == NBW - 1)
    def _():
        pooled = sums_ref[...] / jnp.maximum(cnt_ref[0, :], 1.0)[:, None]
        logits = jnp.dot(pooled, wfc_ref[...],
                         preferred_element_type=jnp.float32) + bfc_ref[...]
        out_ref[...] = jax.nn.log_softmax(logits, axis=1)


def _tc_c(s0, s1, z2, dinv, w2big, b2big, batch3, wfc, bfc):
    return pl.pallas_call(
        _tc_c_body,
        grid=(NBW,),
        in_specs=[
            pl.BlockSpec((1, RW, 128), lambda i: (i, 0, 0)),
            pl.BlockSpec((1, RW, 128), lambda i: (i, 0, 0)),
            pl.BlockSpec((1, RW, 128), lambda i: (i, 0, 0)),
            pl.BlockSpec((1, RW, 128), lambda i: (i, 0, 0)),
            pl.BlockSpec((128, 256), lambda i: (0, 0)),
            pl.BlockSpec((1, 256), lambda i: (0, 0)),
            pl.BlockSpec((1, RW, 8), lambda i: (i, 0, 0)),
            pl.BlockSpec((32, 2), lambda i: (0, 0)),
            pl.BlockSpec((1, 2), lambda i: (0, 0)),
        ],
        out_specs=pl.BlockSpec((G, 2), lambda i: (0, 0)),
        out_shape=jax.ShapeDtypeStruct((G, 2), jnp.float32),
        scratch_shapes=[
            pltpu.VMEM((G, 32), jnp.float32),
            pltpu.VMEM((1, G), jnp.float32),
        ],
    )(s0, s1, z2, dinv, w2big, b2big, batch3, wfc, bfc)


def kernel(x, edge_index, batch, W1, b1, W2, b2, Wfc, bfc):
    src2 = edge_index[0].reshape(NCHUNKS, CH)
    dst2 = edge_index[1].reshape(NCHUNKS, CH)
    zeros = jnp.zeros((RPT, F), jnp.float32)
    ones = jnp.ones((CH, F), jnp.float32)

    eye8 = jnp.eye(8, dtype=jnp.float32)
    w1big = jnp.kron(eye8, W1)
    w2big = jnp.kron(eye8, W2)
    b1big = jnp.tile(b1, 8).reshape(1, 128)
    b2big = jnp.tile(b2, 8).reshape(1, 256)
    xw = x.reshape(NBW, RW, 64)
    batch3 = batch.reshape(NBW, RW, 8)

    hp = _sc_degree(dst2, zeros, ones)
    hpw = hp.reshape(NC, NBW, RW, 128)
    z1, dinv = _tc_a(hpw[0], hpw[1], xw, w1big)
    s1 = _sc_propagate(src2, dst2, z1.reshape(N, F), zeros)
    s1w = s1.reshape(NC, NBW, RW, 128)
    z2 = _tc_b(s1w[0], s1w[1], z1, dinv, b1big)
    s2 = _sc_propagate(src2, dst2, z2.reshape(N, F), zeros)
    s2w = s2.reshape(NC, NBW, RW, 128)
    out = _tc_c(s2w[0], s2w[1], z2, dinv, w2big, b2big, batch3,
                Wfc, bfc.reshape(1, 2))
    return out


# speedup vs baseline: 127.0033x; 1.3194x over previous
import functools

import jax
import jax.numpy as jnp
from jax import lax
from jax.experimental import pallas as pl
from jax.experimental.pallas import tpu as pltpu
from jax.experimental.pallas import tpu_sc as plsc

N = 100000
E = 6400000
G = 128
F = 16

NC = 2
NS = 16
NW = NC * NS
CH = 128
NCHUNKS = E // CH
NFULL = NCHUNKS // NW
NREM = NCHUNKS - NFULL * NW
S = 5
NSUPER = 312
NPIPE = S * NSUPER
NTAIL = NFULL - NPIPE
NPAD = 100032
RPT = NPAD // NS

_mesh = plsc.VectorSubcoreMesh(core_axis_name="c", subcore_axis_name="s")
_sc_params = pltpu.CompilerParams(use_tc_tiling_on_sc=False)


def _zero_acc(zeros_hbm, acc, s):
    pltpu.sync_copy(zeros_hbm, acc.at[pl.ds(s * RPT, RPT)])


def _flush_acc(acc, out_hbm, c, s):
    pltpu.sync_copy(acc.at[pl.ds(s * RPT, RPT)],
                    out_hbm.at[c, pl.ds(s * RPT, RPT)])


def _sc_degree(dst2, zeros, ones):
    @functools.partial(
        pl.kernel,
        out_type=jax.ShapeDtypeStruct((NC, NPAD, F), jnp.float32),
        mesh=_mesh,
        compiler_params=_sc_params,
        scratch_types=[
            pltpu.VMEM((S, CH), jnp.int32),
            pltpu.VMEM((S, CH), jnp.int32),
            pltpu.VMEM((CH, F), jnp.float32),
            pltpu.VMEM_SHARED((NPAD, F), jnp.float32),
            pltpu.SemaphoreType.DMA,
            pltpu.SemaphoreType.DMA,
            pltpu.SemaphoreType.DMA,
            pltpu.SemaphoreType.DMA,
        ],
    )
    def k(dst_hbm, zeros_hbm, ones_hbm, out_hbm,
          didx0, didx1, ones_v, acc, dsem0, dsem1, ssem0, ssem1):
        didx = (didx0, didx1)
        dsem = (dsem0, dsem1)
        ssem = (ssem0, ssem1)
        c = lax.axis_index("c")
        s = lax.axis_index("s")
        wid = c * NS + s
        cw = wid * NFULL + jnp.minimum(wid, NREM)
        pltpu.async_copy(dst_hbm.at[pl.ds(cw, S)], didx0, dsem0)
        _zero_acc(zeros_hbm, acc, s)
        pltpu.sync_copy(ones_hbm, ones_v)
        plsc.subcore_barrier()

        def body(i, p, np):
            pltpu.make_async_copy(dst_hbm.at[pl.ds(cw + i * S, S)],
                                  didx[p], dsem[p]).wait()

            @pl.when(i > 0)
            def _():
                for j in range(S):
                    pltpu.make_async_copy(ones_v, acc.at[didx[np].at[j]],
                                          ssem[np]).wait()

            @pl.when(i + 1 < NSUPER)
            def _():
                pltpu.async_copy(dst_hbm.at[pl.ds(cw + (i + 1) * S, S)],
                                 didx[np], dsem[np])

            for j in range(S):
                pltpu.async_copy(ones_v, acc.at[didx[p].at[j]], ssem[p],
                                 add=True)

        @pl.loop(0, NSUPER, step=2)
        def _(i):
            body(i, 0, 1)
            body(i + 1, 1, 0)

        for j in range(S):
            pltpu.make_async_copy(ones_v, acc.at[didx1.at[j]], ssem1).wait()

        def tail_chunk(t):
            pltpu.sync_copy(dst_hbm.at[pl.ds(cw + NPIPE + t, 1)],
                            didx0.at[pl.ds(0, 1)])
            pltpu.sync_copy(ones_v, acc.at[didx0.at[0]], add=True)

        for t in range(NTAIL):
            tail_chunk(t)

        @pl.when(wid < NREM)
        def _():
            tail_chunk(NTAIL)

        plsc.subcore_barrier()
        _flush_acc(acc, out_hbm, c, s)

    return k(dst2, zeros, ones)


def _sc_propagate(src2, dst2, z, zeros):
    @functools.partial(
        pl.kernel,
        out_type=jax.ShapeDtypeStruct((NC, NPAD, F), jnp.float32),
        mesh=_mesh,
        compiler_params=_sc_params,
        scratch_types=[
            pltpu.VMEM((S, CH), jnp.int32),
            pltpu.VMEM((S, CH), jnp.int32),
            pltpu.VMEM((S, CH), jnp.int32),
            pltpu.VMEM((S, CH), jnp.int32),
            pltpu.VMEM((S, CH, F), jnp.float32),
            pltpu.VMEM((S, CH, F), jnp.float32),
            pltpu.VMEM_SHARED((NPAD, F), jnp.float32),
            pltpu.SemaphoreType.DMA,
            pltpu.SemaphoreType.DMA,
            pltpu.SemaphoreType.DMA,
            pltpu.SemaphoreType.DMA,
            pltpu.SemaphoreType.DMA,
            pltpu.SemaphoreType.DMA,
        ],
    )
    def k(src_hbm, dst_hbm, z_hbm, zeros_hbm, out_hbm,
          sidx0, sidx1, didx0, didx1, rows0, rows1, acc,
          dsem0, dsem1, gsem0, gsem1, ssem0, ssem1):
        sidx = (sidx0, sidx1)
        didx = (didx0, didx1)
        rows = (rows0, rows1)
        dsem = (dsem0, dsem1)
        gsem = (gsem0, gsem1)
        ssem = (ssem0, ssem1)
        c = lax.axis_index("c")
        s = lax.axis_index("s")
        wid = c * NS + s
        cw = wid * NFULL + jnp.minimum(wid, NREM)
        pltpu.async_copy(src_hbm.at[pl.ds(cw, S)], sidx0, dsem0)
        pltpu.async_copy(dst_hbm.at[pl.ds(cw, S)], didx0, dsem0)
        _zero_acc(zeros_hbm, acc, s)
        plsc.subcore_barrier()

        def body(i, p, np):
            pltpu.make_async_copy(src_hbm.at[pl.ds(cw + i * S, S)],
                                  sidx[p], dsem[p]).wait()
            pltpu.make_async_copy(dst_hbm.at[pl.ds(cw + i * S, S)],
                                  didx[p], dsem[p]).wait()
            gh = [pltpu.async_copy(z_hbm.at[sidx[p].at[j]], rows[p].at[j],
                                   gsem[p]) for j in range(S)]

            @pl.when(i > 0)
            def _():
                for j in range(S):
                    pltpu.make_async_copy(rows[np].at[j],
                                          acc.at[didx[np].at[j]],
                                          ssem[np]).wait()

            @pl.when(i + 1 < NSUPER)
            def _():
                pltpu.async_copy(src_hbm.at[pl.ds(cw + (i + 1) * S, S)],
                                 sidx[np], dsem[np])
                pltpu.async_copy(dst_hbm.at[pl.ds(cw + (i + 1) * S, S)],
                                 didx[np], dsem[np])

            for h in gh:
                h.wait()
            for j in range(S):
                pltpu.async_copy(rows[p].at[j], acc.at[didx[p].at[j]],
                                 ssem[p], add=True)

        @pl.loop(0, NSUPER, step=2)
        def _(i):
            body(i, 0, 1)
            body(i + 1, 1, 0)

        for j in range(S):
            pltpu.make_async_copy(rows1.at[j], acc.at[didx1.at[j]],
                                  ssem1).wait()

        def tail_chunk(t):
            pltpu.sync_copy(src_hbm.at[pl.ds(cw + NPIPE + t, 1)],
                            sidx0.at[pl.ds(0, 1)])
            pltpu.sync_copy(dst_hbm.at[pl.ds(cw + NPIPE + t, 1)],
                            didx0.at[pl.ds(0, 1)])
            pltpu.sync_copy(z_hbm.at[sidx0.at[0]], rows0.at[0])
            pltpu.sync_copy(rows0.at[0], acc.at[didx0.at[0]], add=True)

        for t in range(NTAIL):
            tail_chunk(t)

        @pl.when(wid < NREM)
        def _():
            tail_chunk(NTAIL)

        plsc.subcore_barrier()
        _flush_acc(acc, out_hbm, c, s)

    return k(src2, dst2, z, zeros)


NBW = 3
RW = 4168


def _tc_a_body(h_ref, x_ref, w1_ref, z1_ref, dinv_ref):
    deg = h_ref[0, 0] + h_ref[1, 0] + 1.0
    dinv = lax.rsqrt(deg)
    dinv_ref[0] = dinv
    z1_ref[0] = dinv * jnp.dot(x_ref[0], w1_ref[...],
                               preferred_element_type=jnp.float32)


def _tc_a(hp, xw, w1big):
    return pl.pallas_call(
        _tc_a_body,
        grid=(NBW,),
        in_specs=[
            pl.BlockSpec((NC, 1, RW, 128), lambda i: (0, i, 0, 0)),
            pl.BlockSpec((1, RW, 64), lambda i: (i, 0, 0)),
            pl.BlockSpec((64, 128), lambda i: (0, 0)),
        ],
        out_specs=[
            pl.BlockSpec((1, RW, 128), lambda i: (i, 0, 0)),
            pl.BlockSpec((1, RW, 128), lambda i: (i, 0, 0)),
        ],
        out_shape=[
            jax.ShapeDtypeStruct((NBW, RW, 128), jnp.float32),
            jax.ShapeDtypeStruct((NBW, RW, 128), jnp.float32),
        ],
    )(hp, xw, w1big)


def _tc_b_body(s_ref, z1_ref, dinv_ref, b1_ref, z2_ref):
    dinv = dinv_ref[0]
    h1 = dinv * (s_ref[0, 0] + s_ref[1, 0] + z1_ref[0]) + b1_ref[...]
    z2_ref[0] = dinv * jnp.maximum(h1, 0.0)


def _tc_b(sw, z1, dinv, b1big):
    return pl.pallas_call(
        _tc_b_body,
        grid=(NBW,),
        in_specs=[
            pl.BlockSpec((NC, 1, RW, 128), lambda i: (0, i, 0, 0)),
            pl.BlockSpec((1, RW, 128), lambda i: (i, 0, 0)),
            pl.BlockSpec((1, RW, 128), lambda i: (i, 0, 0)),
            pl.BlockSpec((1, 128), lambda i: (0, 0)),
        ],
        out_specs=pl.BlockSpec((1, RW, 128), lambda i: (i, 0, 0)),
        out_shape=jax.ShapeDtypeStruct((NBW, RW, 128), jnp.float32),
    )(sw, z1, dinv, b1big)


def _tc_c_body(s_ref, z2_ref, dinv_ref, w2_ref, b2_ref, batch_ref,
               wfc_ref, bfc_ref, out_ref, sums_ref, cnt_ref):
    i = pl.program_id(0)

    @pl.when(i == 0)
    def _():
        sums_ref[...] = jnp.zeros_like(sums_ref)
        cnt_ref[...] = jnp.zeros_like(cnt_ref)

    tw = dinv_ref[0] * (s_ref[0, 0] + s_ref[1, 0] + z2_ref[0])
    h2w = jnp.maximum(jnp.dot(tw, w2_ref[...],
                              preferred_element_type=jnp.float32)
                      + b2_ref[...], 0.0)
    bt = batch_ref[0]
    for a in range(8):
        mask = (bt[:, a][:, None] == lax.broadcasted_iota(jnp.int32, (RW, G), 1)
                ).astype(jnp.float32)
        sums_ref[...] += lax.dot_general(
            mask, h2w[:, 32 * a:32 * a + 32], (((0,), (0,)), ((), ())),
            preferred_element_type=jnp.float32)
        cnt_ref[...] += jnp.sum(mask, axis=0, keepdims=True)

    @pl.when(i == NBW - 1)
    def _():
        pooled = sums_ref[...] / jnp.maximum(cnt_ref[0, :], 1.0)[:, None]
        logits = jnp.dot(pooled, wfc_ref[...],
                         preferred_element_type=jnp.float32) + bfc_ref[...]
        out_ref[...] = jax.nn.log_softmax(logits, axis=1)


def _tc_c(sw, z2, dinv, w2big, b2big, batch3, wfc, bfc):
    return pl.pallas_call(
        _tc_c_body,
        grid=(NBW,),
        in_specs=[
            pl.BlockSpec((NC, 1, RW, 128), lambda i: (0, i, 0, 0)),
            pl.BlockSpec((1, RW, 128), lambda i: (i, 0, 0)),
            pl.BlockSpec((1, RW, 128), lambda i: (i, 0, 0)),
            pl.BlockSpec((128, 256), lambda i: (0, 0)),
            pl.BlockSpec((1, 256), lambda i: (0, 0)),
            pl.BlockSpec((1, RW, 8), lambda i: (i, 0, 0)),
            pl.BlockSpec((32, 2), lambda i: (0, 0)),
            pl.BlockSpec((1, 2), lambda i: (0, 0)),
        ],
        out_specs=pl.BlockSpec((G, 2), lambda i: (0, 0)),
        out_shape=jax.ShapeDtypeStruct((G, 2), jnp.float32),
        scratch_shapes=[
            pltpu.VMEM((G, 32), jnp.float32),
            pltpu.VMEM((1, G), jnp.float32),
        ],
    )(sw, z2, dinv, w2big, b2big, batch3, wfc, bfc)


def kernel(x, edge_index, batch, W1, b1, W2, b2, Wfc, bfc):
    src2 = edge_index[0].reshape(NCHUNKS, CH)
    dst2 = edge_index[1].reshape(NCHUNKS, CH)
    zeros = jnp.zeros((RPT, F), jnp.float32)
    ones = jnp.ones((CH, F), jnp.float32)

    eye8 = jnp.eye(8, dtype=jnp.float32)
    w1big = jnp.kron(eye8, W1)
    w2big = jnp.kron(eye8, W2)
    b1big = jnp.tile(b1, 8).reshape(1, 128)
    b2big = jnp.tile(b2, 8).reshape(1, 256)
    xw = jnp.concatenate(
        [x, jnp.zeros((NPAD - N, 8), jnp.float32)]).reshape(NBW, RW, 64)
    batch3 = jnp.concatenate(
        [batch, jnp.full((NPAD - N,), G + 7, jnp.int32)]).reshape(NBW, RW, 8)

    hp = _sc_degree(dst2, zeros, ones)
    hpw = hp.reshape(NC, NBW, RW, 128)
    z1, dinv = _tc_a(hpw, xw, w1big)
    s1 = _sc_propagate(src2, dst2, z1.reshape(NPAD, F), zeros)
    s1w = s1.reshape(NC, NBW, RW, 128)
    z2 = _tc_b(s1w, z1, dinv, b1big)
    s2 = _sc_propagate(src2, dst2, z2.reshape(NPAD, F), zeros)
    s2w = s2.reshape(NC, NBW, RW, 128)
    out = _tc_c(s2w, z2, dinv, w2big, b2big, batch3,
                Wfc, bfc.reshape(1, 2))
    return out


# speedup vs baseline: 140.4539x; 1.1059x over previous
import functools

import jax
import jax.numpy as jnp
from jax import lax
from jax.experimental import pallas as pl
from jax.experimental.pallas import tpu as pltpu
from jax.experimental.pallas import tpu_sc as plsc

N = 100000
E = 6400000
G = 128
F = 16

NC = 2
NS = 16
NW = NC * NS
CH = 128
NCHUNKS = E // CH
NFULL = NCHUNKS // NW
NREM = NCHUNKS - NFULL * NW
S = 6
NSUPER = 260
NPIPE = S * NSUPER
NTAIL = NFULL - NPIPE
NPAD = 100032
RPT = NPAD // NS

_mesh = plsc.VectorSubcoreMesh(core_axis_name="c", subcore_axis_name="s")
_sc_params = pltpu.CompilerParams(use_tc_tiling_on_sc=False)


def _zero_acc(zeros_hbm, acc, s):
    pltpu.sync_copy(zeros_hbm, acc.at[pl.ds(s * RPT, RPT)])


def _flush_acc(acc, out_hbm, c, s):
    pltpu.sync_copy(acc.at[pl.ds(s * RPT, RPT)],
                    out_hbm.at[c, pl.ds(s * RPT, RPT)])


def _sc_degree(e3, zeros, ones):
    @functools.partial(
        pl.kernel,
        out_type=jax.ShapeDtypeStruct((NC, NPAD, F), jnp.float32),
        mesh=_mesh,
        compiler_params=_sc_params,
        scratch_types=[
            pltpu.VMEM((1, S, CH), jnp.int32),
            pltpu.VMEM((1, S, CH), jnp.int32),
            pltpu.VMEM((CH, F), jnp.float32),
            pltpu.VMEM_SHARED((NPAD, F), jnp.float32),
            pltpu.SemaphoreType.DMA,
            pltpu.SemaphoreType.DMA,
            pltpu.SemaphoreType.DMA,
            pltpu.SemaphoreType.DMA,
        ],
    )
    def k(e_hbm, zeros_hbm, ones_hbm, out_hbm,
          didx0, didx1, ones_v, acc, dsem0, dsem1, ssem0, ssem1):
        didx = (didx0, didx1)
        dsem = (dsem0, dsem1)
        ssem = (ssem0, ssem1)
        c = lax.axis_index("c")
        s = lax.axis_index("s")
        wid = c * NS + s
        cw = wid * NFULL + jnp.minimum(wid, NREM)
        pltpu.async_copy(e_hbm.at[pl.ds(1, 1), pl.ds(cw, S)], didx0, dsem0)
        _zero_acc(zeros_hbm, acc, s)
        pltpu.sync_copy(ones_hbm, ones_v)
        plsc.subcore_barrier()

        def body(i, p, np):
            pltpu.make_async_copy(e_hbm.at[pl.ds(1, 1), pl.ds(cw + i * S, S)],
                                  didx[p], dsem[p]).wait()

            @pl.when(i > 0)
            def _():
                for j in range(S):
                    pltpu.make_async_copy(ones_v, acc.at[didx[np].at[0, j]],
                                          ssem[np]).wait()

            @pl.when(i + 1 < NSUPER)
            def _():
                pltpu.async_copy(e_hbm.at[pl.ds(1, 1), pl.ds(cw + (i + 1) * S, S)],
                                 didx[np], dsem[np])

            for j in range(S):
                pltpu.async_copy(ones_v, acc.at[didx[p].at[0, j]], ssem[p],
                                 add=True)

        @pl.loop(0, NSUPER, step=2)
        def _(i):
            body(i, 0, 1)
            body(i + 1, 1, 0)

        for j in range(S):
            pltpu.make_async_copy(ones_v, acc.at[didx1.at[0, j]], ssem1).wait()

        def tail_chunk(t):
            pltpu.sync_copy(e_hbm.at[pl.ds(1, 1), pl.ds(cw + NPIPE + t, 1)],
                            didx0.at[pl.ds(0, 1), pl.ds(0, 1)])
            pltpu.sync_copy(ones_v, acc.at[didx0.at[0, 0]], add=True)

        for t in range(NTAIL):
            tail_chunk(t)

        @pl.when(wid < NREM)
        def _():
            tail_chunk(NTAIL)

        plsc.subcore_barrier()
        _flush_acc(acc, out_hbm, c, s)

    return k(e3, zeros, ones)


def _sc_propagate(e3, z, zeros):
    @functools.partial(
        pl.kernel,
        out_type=jax.ShapeDtypeStruct((NC, NPAD, F), jnp.float32),
        mesh=_mesh,
        compiler_params=_sc_params,
        scratch_types=[
            pltpu.VMEM((1, S, CH), jnp.int32),
            pltpu.VMEM((1, S, CH), jnp.int32),
            pltpu.VMEM((1, S, CH), jnp.int32),
            pltpu.VMEM((1, S, CH), jnp.int32),
            pltpu.VMEM((S, CH, F), jnp.float32),
            pltpu.VMEM((S, CH, F), jnp.float32),
            pltpu.VMEM_SHARED((NPAD, F), jnp.float32),
            pltpu.SemaphoreType.DMA,
            pltpu.SemaphoreType.DMA,
            pltpu.SemaphoreType.DMA,
            pltpu.SemaphoreType.DMA,
            pltpu.SemaphoreType.DMA,
            pltpu.SemaphoreType.DMA,
        ],
    )
    def k(e_hbm, z_hbm, zeros_hbm, out_hbm,
          sidx0, sidx1, didx0, didx1, rows0, rows1, acc,
          dsem0, dsem1, gsem0, gsem1, ssem0, ssem1):
        sidx = (sidx0, sidx1)
        didx = (didx0, didx1)
        rows = (rows0, rows1)
        dsem = (dsem0, dsem1)
        gsem = (gsem0, gsem1)
        ssem = (ssem0, ssem1)
        c = lax.axis_index("c")
        s = lax.axis_index("s")
        wid = c * NS + s
        cw = wid * NFULL + jnp.minimum(wid, NREM)
        pltpu.async_copy(e_hbm.at[pl.ds(0, 1), pl.ds(cw, S)], sidx0, dsem0)
        pltpu.async_copy(e_hbm.at[pl.ds(1, 1), pl.ds(cw, S)], didx0, dsem0)
        _zero_acc(zeros_hbm, acc, s)
        plsc.subcore_barrier()

        def body(i, p, np):
            pltpu.make_async_copy(e_hbm.at[pl.ds(0, 1), pl.ds(cw + i * S, S)],
                                  sidx[p], dsem[p]).wait()
            pltpu.make_async_copy(e_hbm.at[pl.ds(1, 1), pl.ds(cw + i * S, S)],
                                  didx[p], dsem[p]).wait()
            gh = [pltpu.async_copy(z_hbm.at[sidx[p].at[0, j]], rows[p].at[j],
                                   gsem[p]) for j in range(S)]

            @pl.when(i > 0)
            def _():
                for j in range(S):
                    pltpu.make_async_copy(rows[np].at[j],
                                          acc.at[didx[np].at[0, j]],
                                          ssem[np]).wait()

            @pl.when(i + 1 < NSUPER)
            def _():
                pltpu.async_copy(e_hbm.at[pl.ds(0, 1), pl.ds(cw + (i + 1) * S, S)],
                                 sidx[np], dsem[np])
                pltpu.async_copy(e_hbm.at[pl.ds(1, 1), pl.ds(cw + (i + 1) * S, S)],
                                 didx[np], dsem[np])

            for h in gh:
                h.wait()
            for j in range(S):
                pltpu.async_copy(rows[p].at[j], acc.at[didx[p].at[0, j]],
                                 ssem[p], add=True)

        @pl.loop(0, NSUPER, step=2)
        def _(i):
            body(i, 0, 1)
            body(i + 1, 1, 0)

        for j in range(S):
            pltpu.make_async_copy(rows1.at[j], acc.at[didx1.at[0, j]],
                                  ssem1).wait()

        def tail_chunk(t):
            pltpu.sync_copy(e_hbm.at[pl.ds(0, 1), pl.ds(cw + NPIPE + t, 1)],
                            sidx0.at[pl.ds(0, 1), pl.ds(0, 1)])
            pltpu.sync_copy(e_hbm.at[pl.ds(1, 1), pl.ds(cw + NPIPE + t, 1)],
                            didx0.at[pl.ds(0, 1), pl.ds(0, 1)])
            pltpu.sync_copy(z_hbm.at[sidx0.at[0, 0]], rows0.at[0])
            pltpu.sync_copy(rows0.at[0], acc.at[didx0.at[0, 0]], add=True)

        for t in range(NTAIL):
            tail_chunk(t)

        @pl.when(wid < NREM)
        def _():
            tail_chunk(NTAIL)

        plsc.subcore_barrier()
        _flush_acc(acc, out_hbm, c, s)

    return k(e3, z, zeros)


NBW = 3
RW = 4168


def _tc_a_body(h_ref, x_ref, w1_ref, z1_ref, dinv_ref):
    deg = h_ref[0, 0] + h_ref[1, 0] + 1.0
    dinv = lax.rsqrt(deg)
    dinv_ref[0] = dinv
    z1_ref[0] = dinv * jnp.dot(x_ref[0], w1_ref[...],
                               preferred_element_type=jnp.float32)


def _tc_a(hp, xw, w1big):
    return pl.pallas_call(
        _tc_a_body,
        grid=(NBW,),
        in_specs=[
            pl.BlockSpec((NC, 1, RW, 128), lambda i: (0, i, 0, 0)),
            pl.BlockSpec((1, RW, 64), lambda i: (i, 0, 0)),
            pl.BlockSpec((64, 128), lambda i: (0, 0)),
        ],
        out_specs=[
            pl.BlockSpec((1, RW, 128), lambda i: (i, 0, 0)),
            pl.BlockSpec((1, RW, 128), lambda i: (i, 0, 0)),
        ],
        out_shape=[
            jax.ShapeDtypeStruct((NBW, RW, 128), jnp.float32),
            jax.ShapeDtypeStruct((NBW, RW, 128), jnp.float32),
        ],
    )(hp, xw, w1big)


def _tc_b_body(s_ref, z1_ref, dinv_ref, b1_ref, z2_ref):
    dinv = dinv_ref[0]
    h1 = dinv * (s_ref[0, 0] + s_ref[1, 0] + z1_ref[0]) + b1_ref[...]
    z2_ref[0] = dinv * jnp.maximum(h1, 0.0)


def _tc_b(sw, z1, dinv, b1big):
    return pl.pallas_call(
        _tc_b_body,
        grid=(NBW,),
        in_specs=[
            pl.BlockSpec((NC, 1, RW, 128), lambda i: (0, i, 0, 0)),
            pl.BlockSpec((1, RW, 128), lambda i: (i, 0, 0)),
            pl.BlockSpec((1, RW, 128), lambda i: (i, 0, 0)),
            pl.BlockSpec((1, 128), lambda i: (0, 0)),
        ],
        out_specs=pl.BlockSpec((1, RW, 128), lambda i: (i, 0, 0)),
        out_shape=jax.ShapeDtypeStruct((NBW, RW, 128), jnp.float32),
    )(sw, z1, dinv, b1big)


def _tc_c_body(s_ref, z2_ref, dinv_ref, w2_ref, b2_ref, batch_ref,
               wfc_ref, bfc_ref, out_ref, sums_ref, cnt_ref):
    i = pl.program_id(0)

    @pl.when(i == 0)
    def _():
        sums_ref[...] = jnp.zeros_like(sums_ref)
        cnt_ref[...] = jnp.zeros_like(cnt_ref)

    tw = dinv_ref[0] * (s_ref[0, 0] + s_ref[1, 0] + z2_ref[0])
    h2w = jnp.maximum(jnp.dot(tw, w2_ref[...],
                              preferred_element_type=jnp.float32)
                      + b2_ref[...], 0.0)
    bt = batch_ref[0]
    for a in range(8):
        mask = (bt[:, a][:, None] == lax.broadcasted_iota(jnp.int32, (RW, G), 1)
                ).astype(jnp.float32)
        sums_ref[...] += lax.dot_general(
            mask, h2w[:, 32 * a:32 * a + 32], (((0,), (0,)), ((), ())),
            preferred_element_type=jnp.float32)
        cnt_ref[...] += jnp.sum(mask, axis=0, keepdims=True)

    @pl.when(i == NBW - 1)
    def _():
        pooled = sums_ref[...] / jnp.maximum(cnt_ref[0, :], 1.0)[:, None]
        logits = jnp.dot(pooled, wfc_ref[...],
                         preferred_element_type=jnp.float32) + bfc_ref[...]
        out_ref[...] = jax.nn.log_softmax(logits, axis=1)


def _tc_c(sw, z2, dinv, w2big, b2big, batch3, wfc, bfc):
    return pl.pallas_call(
        _tc_c_body,
        grid=(NBW,),
        in_specs=[
            pl.BlockSpec((NC, 1, RW, 128), lambda i: (0, i, 0, 0)),
            pl.BlockSpec((1, RW, 128), lambda i: (i, 0, 0)),
            pl.BlockSpec((1, RW, 128), lambda i: (i, 0, 0)),
            pl.BlockSpec((128, 256), lambda i: (0, 0)),
            pl.BlockSpec((1, 256), lambda i: (0, 0)),
            pl.BlockSpec((1, RW, 8), lambda i: (i, 0, 0)),
            pl.BlockSpec((32, 2), lambda i: (0, 0)),
            pl.BlockSpec((1, 2), lambda i: (0, 0)),
        ],
        out_specs=pl.BlockSpec((G, 2), lambda i: (0, 0)),
        out_shape=jax.ShapeDtypeStruct((G, 2), jnp.float32),
        scratch_shapes=[
            pltpu.VMEM((G, 32), jnp.float32),
            pltpu.VMEM((1, G), jnp.float32),
        ],
    )(sw, z2, dinv, w2big, b2big, batch3, wfc, bfc)


def kernel(x, edge_index, batch, W1, b1, W2, b2, Wfc, bfc):
    e3 = edge_index.reshape(2, NCHUNKS, CH)
    zeros = jnp.zeros((RPT, F), jnp.float32)
    ones = jnp.ones((CH, F), jnp.float32)

    eye8 = jnp.eye(8, dtype=jnp.float32)
    w1big = jnp.kron(eye8, W1)
    w2big = jnp.kron(eye8, W2)
    b1big = jnp.tile(b1, 8).reshape(1, 128)
    b2big = jnp.tile(b2, 8).reshape(1, 256)
    xw = jnp.concatenate(
        [x.reshape(N // 8, 64),
         jnp.zeros(((NPAD - N) // 8, 64), jnp.float32)]).reshape(NBW, RW, 64)
    batch3 = jnp.concatenate(
        [batch.reshape(N // 8, 8),
         jnp.full(((NPAD - N) // 8, 8), G + 7, jnp.int32)]).reshape(NBW, RW, 8)

    hp = _sc_degree(e3, zeros, ones)
    hpw = hp.reshape(NC, NBW, RW, 128)
    z1, dinv = _tc_a(hpw, xw, w1big)
    s1 = _sc_propagate(e3, z1.reshape(NPAD, F), zeros)
    s1w = s1.reshape(NC, NBW, RW, 128)
    z2 = _tc_b(s1w, z1, dinv, b1big)
    s2 = _sc_propagate(e3, z2.reshape(NPAD, F), zeros)
    s2w = s2.reshape(NC, NBW, RW, 128)
    out = _tc_c(s2w, z2, dinv, w2big, b2big, batch3,
                Wfc, bfc.reshape(1, 2))
    return out


# speedup vs baseline: 188.2325x; 1.3402x over previous
import functools

import jax
import jax.numpy as jnp
from jax import lax
from jax.experimental import pallas as pl
from jax.experimental.pallas import tpu as pltpu
from jax.experimental.pallas import tpu_sc as plsc

N = 100000
E = 6400000
G = 128
F = 16

NC = 2
NS = 16
NW = NC * NS
CH = 128
NCHUNKS = E // CH
NFULL = NCHUNKS // NW
NREM = NCHUNKS - NFULL * NW
S = 6
NSUPER = 260
NPIPE = S * NSUPER
NTAIL = NFULL - NPIPE
NPAD = 100096
RPT = NPAD // NS

_mesh = plsc.VectorSubcoreMesh(core_axis_name="c", subcore_axis_name="s")
_sc_params = pltpu.CompilerParams(use_tc_tiling_on_sc=False)


def _zero_acc(zeros_hbm, acc, s):
    pltpu.sync_copy(zeros_hbm, acc.at[pl.ds(s * RPT, RPT)])


def _flush_acc(acc, out_hbm, c, s):
    pltpu.sync_copy(acc.at[pl.ds(s * RPT, RPT)],
                    out_hbm.at[c, pl.ds(s * RPT, RPT)])


def _sc_degree(e3, zeros, ones):
    @functools.partial(
        pl.kernel,
        out_type=jax.ShapeDtypeStruct((NC, NPAD, F), jnp.bfloat16),
        mesh=_mesh,
        compiler_params=_sc_params,
        scratch_types=[
            pltpu.VMEM((1, S, CH), jnp.int32),
            pltpu.VMEM((1, S, CH), jnp.int32),
            pltpu.VMEM((CH, F), jnp.bfloat16),
            pltpu.VMEM_SHARED((NPAD, F), jnp.bfloat16),
            pltpu.SemaphoreType.DMA,
            pltpu.SemaphoreType.DMA,
            pltpu.SemaphoreType.DMA,
            pltpu.SemaphoreType.DMA,
        ],
    )
    def k(e_hbm, zeros_hbm, ones_hbm, out_hbm,
          didx0, didx1, ones_v, acc, dsem0, dsem1, ssem0, ssem1):
        didx = (didx0, didx1)
        dsem = (dsem0, dsem1)
        ssem = (ssem0, ssem1)
        c = lax.axis_index("c")
        s = lax.axis_index("s")
        wid = c * NS + s
        cw = wid * NFULL + jnp.minimum(wid, NREM)
        pltpu.async_copy(e_hbm.at[pl.ds(1, 1), pl.ds(cw, S)], didx0, dsem0)
        _zero_acc(zeros_hbm, acc, s)
        pltpu.sync_copy(ones_hbm, ones_v)
        plsc.subcore_barrier()

        def body(i, p, np):
            pltpu.make_async_copy(e_hbm.at[pl.ds(1, 1), pl.ds(cw + i * S, S)],
                                  didx[p], dsem[p]).wait()

            @pl.when(i > 0)
            def _():
                for j in range(S):
                    pltpu.make_async_copy(ones_v, acc.at[didx[np].at[0, j]],
                                          ssem[np]).wait()

            @pl.when(i + 1 < NSUPER)
            def _():
                pltpu.async_copy(e_hbm.at[pl.ds(1, 1), pl.ds(cw + (i + 1) * S, S)],
                                 didx[np], dsem[np])

            for j in range(S):
                pltpu.async_copy(ones_v, acc.at[didx[p].at[0, j]], ssem[p],
                                 add=True)

        @pl.loop(0, NSUPER, step=2)
        def _(i):
            body(i, 0, 1)
            body(i + 1, 1, 0)

        for j in range(S):
            pltpu.make_async_copy(ones_v, acc.at[didx1.at[0, j]], ssem1).wait()

        def tail_chunk(t):
            pltpu.sync_copy(e_hbm.at[pl.ds(1, 1), pl.ds(cw + NPIPE + t, 1)],
                            didx0.at[pl.ds(0, 1), pl.ds(0, 1)])
            pltpu.sync_copy(ones_v, acc.at[didx0.at[0, 0]], add=True)

        for t in range(NTAIL):
            tail_chunk(t)

        @pl.when(wid < NREM)
        def _():
            tail_chunk(NTAIL)

        plsc.subcore_barrier()
        _flush_acc(acc, out_hbm, c, s)

    return k(e3, zeros, ones)


def _sc_propagate(e3, z, zeros):
    @functools.partial(
        pl.kernel,
        out_type=jax.ShapeDtypeStruct((NC, NPAD, F), jnp.bfloat16),
        mesh=_mesh,
        compiler_params=_sc_params,
        scratch_types=[
            pltpu.VMEM((1, S, CH), jnp.int32),
            pltpu.VMEM((1, S, CH), jnp.int32),
            pltpu.VMEM((1, S, CH), jnp.int32),
            pltpu.VMEM((1, S, CH), jnp.int32),
            pltpu.VMEM((S, CH, F), jnp.bfloat16),
            pltpu.VMEM((S, CH, F), jnp.bfloat16),
            pltpu.VMEM_SHARED((NPAD, F), jnp.bfloat16),
            pltpu.VMEM_SHARED((NPAD, F), jnp.bfloat16),
            pltpu.SemaphoreType.DMA,
            pltpu.SemaphoreType.DMA,
            pltpu.SemaphoreType.DMA,
            pltpu.SemaphoreType.DMA,
            pltpu.SemaphoreType.DMA,
            pltpu.SemaphoreType.DMA,
        ],
    )
    def k(e_hbm, z_hbm, zeros_hbm, out_hbm,
          sidx0, sidx1, didx0, didx1, rows0, rows1, acc, tbl,
          dsem0, dsem1, gsem0, gsem1, ssem0, ssem1):
        sidx = (sidx0, sidx1)
        didx = (didx0, didx1)
        rows = (rows0, rows1)
        dsem = (dsem0, dsem1)
        gsem = (gsem0, gsem1)
        ssem = (ssem0, ssem1)
        c = lax.axis_index("c")
        s = lax.axis_index("s")
        wid = c * NS + s
        cw = wid * NFULL + jnp.minimum(wid, NREM)
        pltpu.async_copy(e_hbm.at[pl.ds(0, 1), pl.ds(cw, S)], sidx0, dsem0)
        pltpu.async_copy(e_hbm.at[pl.ds(1, 1), pl.ds(cw, S)], didx0, dsem0)
        pltpu.sync_copy(z_hbm.at[pl.ds(s * RPT, RPT)], tbl.at[pl.ds(s * RPT, RPT)])
        _zero_acc(zeros_hbm, acc, s)
        plsc.subcore_barrier()

        def body(i, p, np):
            pltpu.make_async_copy(e_hbm.at[pl.ds(0, 1), pl.ds(cw + i * S, S)],
                                  sidx[p], dsem[p]).wait()
            pltpu.make_async_copy(e_hbm.at[pl.ds(1, 1), pl.ds(cw + i * S, S)],
                                  didx[p], dsem[p]).wait()
            gh = [pltpu.async_copy(tbl.at[sidx[p].at[0, j]], rows[p].at[j],
                                   gsem[p]) for j in range(S)]

            @pl.when(i > 0)
            def _():
                for j in range(S):
                    pltpu.make_async_copy(rows[np].at[j],
                                          acc.at[didx[np].at[0, j]],
                                          ssem[np]).wait()

            @pl.when(i + 1 < NSUPER)
            def _():
                pltpu.async_copy(e_hbm.at[pl.ds(0, 1), pl.ds(cw + (i + 1) * S, S)],
                                 sidx[np], dsem[np])
                pltpu.async_copy(e_hbm.at[pl.ds(1, 1), pl.ds(cw + (i + 1) * S, S)],
                                 didx[np], dsem[np])

            for h in gh:
                h.wait()
            for j in range(S):
                pltpu.async_copy(rows[p].at[j], acc.at[didx[p].at[0, j]],
                                 ssem[p], add=True)

        @pl.loop(0, NSUPER, step=2)
        def _(i):
            body(i, 0, 1)
            body(i + 1, 1, 0)

        for j in range(S):
            pltpu.make_async_copy(rows1.at[j], acc.at[didx1.at[0, j]],
                                  ssem1).wait()

        def tail_chunk(t):
            pltpu.sync_copy(e_hbm.at[pl.ds(0, 1), pl.ds(cw + NPIPE + t, 1)],
                            sidx0.at[pl.ds(0, 1), pl.ds(0, 1)])
            pltpu.sync_copy(e_hbm.at[pl.ds(1, 1), pl.ds(cw + NPIPE + t, 1)],
                            didx0.at[pl.ds(0, 1), pl.ds(0, 1)])
            pltpu.sync_copy(tbl.at[sidx0.at[0, 0]], rows0.at[0])
            pltpu.sync_copy(rows0.at[0], acc.at[didx0.at[0, 0]], add=True)

        for t in range(NTAIL):
            tail_chunk(t)

        @pl.when(wid < NREM)
        def _():
            tail_chunk(NTAIL)

        plsc.subcore_barrier()
        _flush_acc(acc, out_hbm, c, s)

    return k(e3, z, zeros)


NBW = 2
RW = 6256


def _tc_a_body(h_ref, x_ref, w1_ref, z1_ref, dinv_ref):
    deg = (h_ref[0, 0].astype(jnp.float32) + h_ref[1, 0].astype(jnp.float32)
           + 1.0)
    dinv = lax.rsqrt(deg)
    dinv_ref[0] = dinv
    z1_ref[0] = (dinv * jnp.dot(x_ref[0], w1_ref[...],
                                preferred_element_type=jnp.float32)
                 ).astype(jnp.bfloat16)


def _tc_a(hp, xw, w1big):
    return pl.pallas_call(
        _tc_a_body,
        grid=(NBW,),
        in_specs=[
            pl.BlockSpec((NC, 1, RW, 128), lambda i: (0, i, 0, 0)),
            pl.BlockSpec((1, RW, 64), lambda i: (i, 0, 0)),
            pl.BlockSpec((64, 128), lambda i: (0, 0)),
        ],
        out_specs=[
            pl.BlockSpec((1, RW, 128), lambda i: (i, 0, 0)),
            pl.BlockSpec((1, RW, 128), lambda i: (i, 0, 0)),
        ],
        out_shape=[
            jax.ShapeDtypeStruct((NBW, RW, 128), jnp.bfloat16),
            jax.ShapeDtypeStruct((NBW, RW, 128), jnp.float32),
        ],
    )(hp, xw, w1big)


def _tc_b_body(s_ref, z1_ref, dinv_ref, b1_ref, z2_ref):
    dinv = dinv_ref[0]
    msum = (s_ref[0, 0].astype(jnp.float32) + s_ref[1, 0].astype(jnp.float32)
            + z1_ref[0].astype(jnp.float32))
    h1 = dinv * msum + b1_ref[...]
    z2_ref[0] = (dinv * jnp.maximum(h1, 0.0)).astype(jnp.bfloat16)


def _tc_b(sw, z1, dinv, b1big):
    return pl.pallas_call(
        _tc_b_body,
        grid=(NBW,),
        in_specs=[
            pl.BlockSpec((NC, 1, RW, 128), lambda i: (0, i, 0, 0)),
            pl.BlockSpec((1, RW, 128), lambda i: (i, 0, 0)),
            pl.BlockSpec((1, RW, 128), lambda i: (i, 0, 0)),
            pl.BlockSpec((1, 128), lambda i: (0, 0)),
        ],
        out_specs=pl.BlockSpec((1, RW, 128), lambda i: (i, 0, 0)),
        out_shape=jax.ShapeDtypeStruct((NBW, RW, 128), jnp.bfloat16),
    )(sw, z1, dinv, b1big)


def _tc_c_body(s_ref, z2_ref, dinv_ref, w2_ref, b2_ref, batch_ref,
               wfc_ref, bfc_ref, out_ref, sums_ref, cnt_ref):
    i = pl.program_id(0)

    @pl.when(i == 0)
    def _():
        sums_ref[...] = jnp.zeros_like(sums_ref)
        cnt_ref[...] = jnp.zeros_like(cnt_ref)

    tw = dinv_ref[0] * (s_ref[0, 0].astype(jnp.float32)
                        + s_ref[1, 0].astype(jnp.float32)
                        + z2_ref[0].astype(jnp.float32))
    h2w = jnp.maximum(jnp.dot(tw, w2_ref[...],
                              preferred_element_type=jnp.float32)
                      + b2_ref[...], 0.0)
    bt = batch_ref[0]
    for a in range(8):
        mask = (bt[:, a][:, None] == lax.broadcasted_iota(jnp.int32, (RW, G), 1)
                ).astype(jnp.float32)
        sums_ref[...] += lax.dot_general(
            mask, h2w[:, 32 * a:32 * a + 32], (((0,), (0,)), ((), ())),
            preferred_element_type=jnp.float32)
        cnt_ref[...] += jnp.sum(mask, axis=0, keepdims=True)

    @pl.when(i == NBW - 1)
    def _():
        pooled = sums_ref[...] / jnp.maximum(cnt_ref[0, :], 1.0)[:, None]
        logits = jnp.dot(pooled, wfc_ref[...],
                         preferred_element_type=jnp.float32) + bfc_ref[...]
        out_ref[...] = jax.nn.log_softmax(logits, axis=1)


def _tc_c(sw, z2, dinv, w2big, b2big, batch3, wfc, bfc):
    return pl.pallas_call(
        _tc_c_body,
        grid=(NBW,),
        in_specs=[
            pl.BlockSpec((NC, 1, RW, 128), lambda i: (0, i, 0, 0)),
            pl.BlockSpec((1, RW, 128), lambda i: (i, 0, 0)),
            pl.BlockSpec((1, RW, 128), lambda i: (i, 0, 0)),
            pl.BlockSpec((128, 256), lambda i: (0, 0)),
            pl.BlockSpec((1, 256), lambda i: (0, 0)),
            pl.BlockSpec((1, RW, 8), lambda i: (i, 0, 0)),
            pl.BlockSpec((32, 2), lambda i: (0, 0)),
            pl.BlockSpec((1, 2), lambda i: (0, 0)),
        ],
        out_specs=pl.BlockSpec((G, 2), lambda i: (0, 0)),
        out_shape=jax.ShapeDtypeStruct((G, 2), jnp.float32),
        scratch_shapes=[
            pltpu.VMEM((G, 32), jnp.float32),
            pltpu.VMEM((1, G), jnp.float32),
        ],
    )(sw, z2, dinv, w2big, b2big, batch3, wfc, bfc)


def kernel(x, edge_index, batch, W1, b1, W2, b2, Wfc, bfc):
    e3 = edge_index.reshape(2, NCHUNKS, CH)
    zeros = jnp.zeros((RPT, F), jnp.bfloat16)
    ones = jnp.ones((CH, F), jnp.bfloat16)

    eye8 = jnp.eye(8, dtype=jnp.float32)
    w1big = jnp.kron(eye8, W1)
    w2big = jnp.kron(eye8, W2)
    b1big = jnp.tile(b1, 8).reshape(1, 128)
    b2big = jnp.tile(b2, 8).reshape(1, 256)
    xw = jnp.concatenate(
        [x.reshape(N // 8, 64),
         jnp.zeros(((NPAD - N) // 8, 64), jnp.float32)]).reshape(NBW, RW, 64)
    batch3 = jnp.concatenate(
        [batch.reshape(N // 8, 8),
         jnp.full(((NPAD - N) // 8, 8), G + 7, jnp.int32)]).reshape(NBW, RW, 8)

    hp = _sc_degree(e3, zeros, ones)
    hpw = hp.reshape(NC, NBW, RW, 128)
    z1, dinv = _tc_a(hpw, xw, w1big)
    s1 = _sc_propagate(e3, z1.reshape(NPAD, F), zeros)
    s1w = s1.reshape(NC, NBW, RW, 128)
    z2 = _tc_b(s1w, z1, dinv, b1big)
    s2 = _sc_propagate(e3, z2.reshape(NPAD, F), zeros)
    s2w = s2.reshape(NC, NBW, RW, 128)
    out = _tc_c(s2w, z2, dinv, w2big, b2big, batch3,
                Wfc, bfc.reshape(1, 2))
    return out


# speedup vs baseline: 213.2288x; 1.1328x over previous
import functools

import jax
import jax.numpy as jnp
from jax import lax
from jax.experimental import pallas as pl
from jax.experimental.pallas import tpu as pltpu
from jax.experimental.pallas import tpu_sc as plsc

N = 100000
E = 6400000
G = 128
F = 16

NC = 2
NS = 16
NW = NC * NS
CH = 128
NCHUNKS = E // CH
NFULL = NCHUNKS // NW
NREM = NCHUNKS - NFULL * NW
S = 10
NSUPER = 156
NPIPE = S * NSUPER
NTAIL = NFULL - NPIPE
NPAD = 100096
RPT = NPAD // NS

_mesh = plsc.VectorSubcoreMesh(core_axis_name="c", subcore_axis_name="s")
_sc_params = pltpu.CompilerParams(use_tc_tiling_on_sc=False)


def _zero_acc(zeros_hbm, acc, s):
    pltpu.sync_copy(zeros_hbm, acc.at[pl.ds(s * RPT, RPT)])


def _flush_acc(acc, out_hbm, c, s):
    pltpu.sync_copy(acc.at[pl.ds(s * RPT, RPT)],
                    out_hbm.at[c, pl.ds(s * RPT, RPT)])


def _sc_degree(e3, zeros, ones):
    @functools.partial(
        pl.kernel,
        out_type=jax.ShapeDtypeStruct((NC, NPAD, F), jnp.bfloat16),
        mesh=_mesh,
        compiler_params=_sc_params,
        scratch_types=[
            pltpu.VMEM((1, S, CH), jnp.int32),
            pltpu.VMEM((1, S, CH), jnp.int32),
            pltpu.VMEM((CH, F), jnp.bfloat16),
            pltpu.VMEM_SHARED((NPAD, F), jnp.bfloat16),
            pltpu.SemaphoreType.DMA,
            pltpu.SemaphoreType.DMA,
            pltpu.SemaphoreType.DMA,
            pltpu.SemaphoreType.DMA,
        ],
    )
    def k(e_hbm, zeros_hbm, ones_hbm, out_hbm,
          didx0, didx1, ones_v, acc, dsem0, dsem1, ssem0, ssem1):
        didx = (didx0, didx1)
        dsem = (dsem0, dsem1)
        ssem = (ssem0, ssem1)
        c = lax.axis_index("c")
        s = lax.axis_index("s")
        wid = c * NS + s
        cw = wid * NFULL + jnp.minimum(wid, NREM)
        pltpu.async_copy(e_hbm.at[pl.ds(1, 1), pl.ds(cw, S)], didx0, dsem0)
        _zero_acc(zeros_hbm, acc, s)
        pltpu.sync_copy(ones_hbm, ones_v)
        plsc.subcore_barrier()

        def body(i, p, np):
            pltpu.make_async_copy(e_hbm.at[pl.ds(1, 1), pl.ds(cw + i * S, S)],
                                  didx[p], dsem[p]).wait()

            @pl.when(i > 0)
            def _():
                for j in range(S):
                    pltpu.make_async_copy(ones_v, acc.at[didx[np].at[0, j]],
                                          ssem[np]).wait()

            @pl.when(i + 1 < NSUPER)
            def _():
                pltpu.async_copy(e_hbm.at[pl.ds(1, 1), pl.ds(cw + (i + 1) * S, S)],
                                 didx[np], dsem[np])

            for j in range(S):
                pltpu.async_copy(ones_v, acc.at[didx[p].at[0, j]], ssem[p],
                                 add=True)

        @pl.loop(0, NSUPER, step=2)
        def _(i):
            body(i, 0, 1)
            body(i + 1, 1, 0)

        for j in range(S):
            pltpu.make_async_copy(ones_v, acc.at[didx1.at[0, j]], ssem1).wait()

        def tail_chunk(t):
            pltpu.sync_copy(e_hbm.at[pl.ds(1, 1), pl.ds(cw + NPIPE + t, 1)],
                            didx0.at[pl.ds(0, 1), pl.ds(0, 1)])
            pltpu.sync_copy(ones_v, acc.at[didx0.at[0, 0]], add=True)

        for t in range(NTAIL):
            tail_chunk(t)

        @pl.when(wid < NREM)
        def _():
            tail_chunk(NTAIL)

        plsc.subcore_barrier()
        _flush_acc(acc, out_hbm, c, s)

    return k(e3, zeros, ones)


def _sc_propagate(e3, z, zeros):
    @functools.partial(
        pl.kernel,
        out_type=jax.ShapeDtypeStruct((NC, NPAD, F), jnp.bfloat16),
        mesh=_mesh,
        compiler_params=_sc_params,
        scratch_types=[
            pltpu.VMEM((1, S, CH), jnp.int32),
            pltpu.VMEM((1, S, CH), jnp.int32),
            pltpu.VMEM((1, S, CH), jnp.int32),
            pltpu.VMEM((1, S, CH), jnp.int32),
            pltpu.VMEM((S, CH, F), jnp.bfloat16),
            pltpu.VMEM((S, CH, F), jnp.bfloat16),
            pltpu.VMEM_SHARED((NPAD, F), jnp.bfloat16),
            pltpu.VMEM_SHARED((NPAD, F), jnp.bfloat16),
            pltpu.SemaphoreType.DMA,
            pltpu.SemaphoreType.DMA,
            pltpu.SemaphoreType.DMA,
            pltpu.SemaphoreType.DMA,
            pltpu.SemaphoreType.DMA,
            pltpu.SemaphoreType.DMA,
        ],
    )
    def k(e_hbm, z_hbm, zeros_hbm, out_hbm,
          sidx0, sidx1, didx0, didx1, rows0, rows1, acc, tbl,
          dsem0, dsem1, gsem0, gsem1, ssem0, ssem1):
        sidx = (sidx0, sidx1)
        didx = (didx0, didx1)
        rows = (rows0, rows1)
        dsem = (dsem0, dsem1)
        gsem = (gsem0, gsem1)
        ssem = (ssem0, ssem1)
        c = lax.axis_index("c")
        s = lax.axis_index("s")
        wid = c * NS + s
        cw = wid * NFULL + jnp.minimum(wid, NREM)
        pltpu.async_copy(e_hbm.at[pl.ds(0, 1), pl.ds(cw, S)], sidx0, dsem0)
        pltpu.async_copy(e_hbm.at[pl.ds(1, 1), pl.ds(cw, S)], didx0, dsem0)
        pltpu.sync_copy(z_hbm.at[pl.ds(s * RPT, RPT)], tbl.at[pl.ds(s * RPT, RPT)])
        _zero_acc(zeros_hbm, acc, s)
        plsc.subcore_barrier()

        def body(i, p, np):
            pltpu.make_async_copy(e_hbm.at[pl.ds(0, 1), pl.ds(cw + i * S, S)],
                                  sidx[p], dsem[p]).wait()
            pltpu.make_async_copy(e_hbm.at[pl.ds(1, 1), pl.ds(cw + i * S, S)],
                                  didx[p], dsem[p]).wait()
            gh = [pltpu.async_copy(tbl.at[sidx[p].at[0, j]], rows[p].at[j],
                                   gsem[p]) for j in range(S)]

            @pl.when(i > 0)
            def _():
                for j in range(S):
                    pltpu.make_async_copy(rows[np].at[j],
                                          acc.at[didx[np].at[0, j]],
                                          ssem[np]).wait()

            @pl.when(i + 1 < NSUPER)
            def _():
                pltpu.async_copy(e_hbm.at[pl.ds(0, 1), pl.ds(cw + (i + 1) * S, S)],
                                 sidx[np], dsem[np])
                pltpu.async_copy(e_hbm.at[pl.ds(1, 1), pl.ds(cw + (i + 1) * S, S)],
                                 didx[np], dsem[np])

            for h in gh:
                h.wait()
            for j in range(S):
                pltpu.async_copy(rows[p].at[j], acc.at[didx[p].at[0, j]],
                                 ssem[p], add=True)

        @pl.loop(0, NSUPER, step=2)
        def _(i):
            body(i, 0, 1)
            body(i + 1, 1, 0)

        for j in range(S):
            pltpu.make_async_copy(rows1.at[j], acc.at[didx1.at[0, j]],
                                  ssem1).wait()

        def tail_chunk(t):
            pltpu.sync_copy(e_hbm.at[pl.ds(0, 1), pl.ds(cw + NPIPE + t, 1)],
                            sidx0.at[pl.ds(0, 1), pl.ds(0, 1)])
            pltpu.sync_copy(e_hbm.at[pl.ds(1, 1), pl.ds(cw + NPIPE + t, 1)],
                            didx0.at[pl.ds(0, 1), pl.ds(0, 1)])
            pltpu.sync_copy(tbl.at[sidx0.at[0, 0]], rows0.at[0])
            pltpu.sync_copy(rows0.at[0], acc.at[didx0.at[0, 0]], add=True)

        for t in range(NTAIL):
            tail_chunk(t)

        @pl.when(wid < NREM)
        def _():
            tail_chunk(NTAIL)

        plsc.subcore_barrier()
        _flush_acc(acc, out_hbm, c, s)

    return k(e3, z, zeros)


NBW = 2
RW = 6256


def _tc_a_body(h_ref, x_ref, w1_ref, z1_ref, dinv_ref):
    deg = (h_ref[0, 0].astype(jnp.float32) + h_ref[1, 0].astype(jnp.float32)
           + 1.0)
    dinv = lax.rsqrt(deg)
    dinv_ref[0] = dinv
    z1_ref[0] = (dinv * jnp.dot(x_ref[0], w1_ref[...],
                                preferred_element_type=jnp.float32)
                 ).astype(jnp.bfloat16)


def _tc_a(hp, xw, w1big):
    return pl.pallas_call(
        _tc_a_body,
        grid=(NBW,),
        in_specs=[
            pl.BlockSpec((NC, 1, RW, 128), lambda i: (0, i, 0, 0)),
            pl.BlockSpec((1, RW, 64), lambda i: (i, 0, 0)),
            pl.BlockSpec((64, 128), lambda i: (0, 0)),
        ],
        out_specs=[
            pl.BlockSpec((1, RW, 128), lambda i: (i, 0, 0)),
            pl.BlockSpec((1, RW, 128), lambda i: (i, 0, 0)),
        ],
        out_shape=[
            jax.ShapeDtypeStruct((NBW, RW, 128), jnp.bfloat16),
            jax.ShapeDtypeStruct((NBW, RW, 128), jnp.float32),
        ],
    )(hp, xw, w1big)


def _tc_b_body(s_ref, z1_ref, dinv_ref, b1_ref, z2_ref):
    dinv = dinv_ref[0]
    msum = (s_ref[0, 0].astype(jnp.float32) + s_ref[1, 0].astype(jnp.float32)
            + z1_ref[0].astype(jnp.float32))
    h1 = dinv * msum + b1_ref[...]
    z2_ref[0] = (dinv * jnp.maximum(h1, 0.0)).astype(jnp.bfloat16)


def _tc_b(sw, z1, dinv, b1big):
    return pl.pallas_call(
        _tc_b_body,
        grid=(NBW,),
        in_specs=[
            pl.BlockSpec((NC, 1, RW, 128), lambda i: (0, i, 0, 0)),
            pl.BlockSpec((1, RW, 128), lambda i: (i, 0, 0)),
            pl.BlockSpec((1, RW, 128), lambda i: (i, 0, 0)),
            pl.BlockSpec((1, 128), lambda i: (0, 0)),
        ],
        out_specs=pl.BlockSpec((1, RW, 128), lambda i: (i, 0, 0)),
        out_shape=jax.ShapeDtypeStruct((NBW, RW, 128), jnp.bfloat16),
    )(sw, z1, dinv, b1big)


def _tc_c_body(s_ref, z2_ref, dinv_ref, w2_ref, b2_ref, batch_ref,
               wfc_ref, bfc_ref, out_ref, sums_ref, cnt_ref):
    i = pl.program_id(0)

    @pl.when(i == 0)
    def _():
        sums_ref[...] = jnp.zeros_like(sums_ref)
        cnt_ref[...] = jnp.zeros_like(cnt_ref)

    tw = dinv_ref[0] * (s_ref[0, 0].astype(jnp.float32)
                        + s_ref[1, 0].astype(jnp.float32)
                        + z2_ref[0].astype(jnp.float32))
    h2w = jnp.maximum(jnp.dot(tw, w2_ref[...],
                              preferred_element_type=jnp.float32)
                      + b2_ref[...], 0.0)
    bt = batch_ref[0]
    for a in range(8):
        mask = (bt[:, a][:, None] == lax.broadcasted_iota(jnp.int32, (RW, G), 1)
                ).astype(jnp.float32)
        sums_ref[...] += lax.dot_general(
            mask, h2w[:, 32 * a:32 * a + 32], (((0,), (0,)), ((), ())),
            preferred_element_type=jnp.float32)
        cnt_ref[...] += jnp.sum(mask, axis=0, keepdims=True)

    @pl.when(i == NBW - 1)
    def _():
        pooled = sums_ref[...] / jnp.maximum(cnt_ref[0, :], 1.0)[:, None]
        logits = jnp.dot(pooled, wfc_ref[...],
                         preferred_element_type=jnp.float32) + bfc_ref[...]
        out_ref[...] = jax.nn.log_softmax(logits, axis=1)


def _tc_c(sw, z2, dinv, w2big, b2big, batch3, wfc, bfc):
    return pl.pallas_call(
        _tc_c_body,
        grid=(NBW,),
        in_specs=[
            pl.BlockSpec((NC, 1, RW, 128), lambda i: (0, i, 0, 0)),
            pl.BlockSpec((1, RW, 128), lambda i: (i, 0, 0)),
            pl.BlockSpec((1, RW, 128), lambda i: (i, 0, 0)),
            pl.BlockSpec((128, 256), lambda i: (0, 0)),
            pl.BlockSpec((1, 256), lambda i: (0, 0)),
            pl.BlockSpec((1, RW, 8), lambda i: (i, 0, 0)),
            pl.BlockSpec((32, 2), lambda i: (0, 0)),
            pl.BlockSpec((1, 2), lambda i: (0, 0)),
        ],
        out_specs=pl.BlockSpec((G, 2), lambda i: (0, 0)),
        out_shape=jax.ShapeDtypeStruct((G, 2), jnp.float32),
        scratch_shapes=[
            pltpu.VMEM((G, 32), jnp.float32),
            pltpu.VMEM((1, G), jnp.float32),
        ],
    )(sw, z2, dinv, w2big, b2big, batch3, wfc, bfc)


def kernel(x, edge_index, batch, W1, b1, W2, b2, Wfc, bfc):
    e3 = edge_index.reshape(2, NCHUNKS, CH)
    zeros = jnp.zeros((RPT, F), jnp.bfloat16)
    ones = jnp.ones((CH, F), jnp.bfloat16)

    eye8 = jnp.eye(8, dtype=jnp.float32)
    w1big = jnp.kron(eye8, W1)
    w2big = jnp.kron(eye8, W2)
    b1big = jnp.tile(b1, 8).reshape(1, 128)
    b2big = jnp.tile(b2, 8).reshape(1, 256)
    xw = jnp.concatenate(
        [x.reshape(N // 8, 64),
         jnp.zeros(((NPAD - N) // 8, 64), jnp.float32)]).reshape(NBW, RW, 64)
    batch3 = jnp.concatenate(
        [batch.reshape(N // 8, 8),
         jnp.full(((NPAD - N) // 8, 8), G + 7, jnp.int32)]).reshape(NBW, RW, 8)

    hp = _sc_degree(e3, zeros, ones)
    hpw = hp.reshape(NC, NBW, RW, 128)
    z1, dinv = _tc_a(hpw, xw, w1big)
    s1 = _sc_propagate(e3, z1.reshape(NPAD, F), zeros)
    s1w = s1.reshape(NC, NBW, RW, 128)
    z2 = _tc_b(s1w, z1, dinv, b1big)
    s2 = _sc_propagate(e3, z2.reshape(NPAD, F), zeros)
    s2w = s2.reshape(NC, NBW, RW, 128)
    out = _tc_c(s2w, z2, dinv, w2big, b2big, batch3,
                Wfc, bfc.reshape(1, 2))
    return out
